# baseline (speedup 1.0000x reference)
import jax
import jax.numpy as jnp
from jax import lax
from jax.experimental import pallas as pl
from jax.experimental.pallas import tpu as pltpu
from jax.experimental.pallas import tpu_sc as plsc

N_POINTS = 200000
N_PX = 512
N_PY = 512
IN_F = 6
OUT_F = 64
EPS = 1e-5

BK = 1024
P_PAD = 200704
N_BLOCKS = P_PAD // BK

NCORE = 2
NSUB = 16
PT = P_PAD // NSUB
CHUNK = 8192
SHIFT = 13
NBUCKETS = (N_PX * N_PY) // CHUNK
PASSES = NBUCKETS // NCORE
WAVE = 128
LIST_CAP = PT + NBUCKETS * WAVE
ROWS_PER_SUB = CHUNK // NSUB
DUMMY_CELL = CHUNK


def _stats_kernel(x_ref, sx_ref, sxx_ref):
    i = pl.program_id(0)
    xb = x_ref[...]
    sx = jnp.sum(xb, axis=0, keepdims=True)
    sxx = lax.dot_general(xb, xb, (((0,), (0,)), ((), ())),
                          preferred_element_type=jnp.float32)

    @pl.when(i == 0)
    def _():
        sx_ref[...] = jnp.zeros_like(sx_ref)
        sxx_ref[...] = jnp.zeros_like(sxx_ref)

    sx_ref[...] += sx
    sxx_ref[...] += sxx


def _forward_kernel(x_ref, sx_ref, sxx_ref, wt_ref, b_ref, g_ref, beta_ref,
                    h_ref):
    mu = sx_ref[...] / N_POINTS
    m2 = sxx_ref[...] / N_POINTS
    wt = wt_ref[...]
    b = b_ref[...]
    mean = lax.dot_general(mu, wt, (((1,), (0,)), ((), ())),
                           preferred_element_type=jnp.float32) + b
    m2w = lax.dot_general(m2, wt, (((1,), (0,)), ((), ())),
                          preferred_element_type=jnp.float32)
    quad = jnp.sum(wt * m2w, axis=0, keepdims=True)
    var = quad + 2.0 * b * mean - b * b - mean * mean
    s = g_ref[...] * lax.rsqrt(var + EPS)
    b2 = (b - mean) * s + beta_ref[...]
    h = lax.dot_general(x_ref[...], wt, (((1,), (0,)), ((), ())),
                        preferred_element_type=jnp.float32)
    h_ref[...] = jnp.maximum(h * s + b2, 0.0)


def _scatter_kernel(xi_hbm, yi_hbm, h_hbm, out_hbm,
                    sc_list, sp_list, flat_v, hist, tails, starts, bcount,
                    cell_buf, pid_buf, rows_v, zbuf, chunk):
    c = lax.axis_index("c")
    s = lax.axis_index("s")
    base = s * PT
    iota = lax.iota(jnp.int32, 16)
    zeros16i = jnp.zeros((16,), jnp.int32)
    ones_m = iota < 16
    lanebase = iota * NBUCKETS

    pltpu.sync_copy(xi_hbm.at[pl.ds(base, PT)], sc_list.at[pl.ds(0, PT)])
    pltpu.sync_copy(yi_hbm.at[pl.ds(base, PT)], sp_list.at[pl.ds(0, PT)])

    def _flat_body(i, carry):
        xv = sc_list[pl.ds(i * 16, 16)]
        yv = sp_list[pl.ds(i * 16, 16)]
        flat_v[pl.ds(i * 16, 16)] = xv * N_PY + yv
        return carry

    lax.fori_loop(0, PT // 16, _flat_body, 0)

    zeros16f = jnp.zeros((16,), jnp.float32)

    def _zero_body(i, carry):
        for j in range(OUT_F // 16):
            zbuf[i, pl.ds(j * 16, 16)] = zeros16f
        return carry

    lax.fori_loop(0, zbuf.shape[0], _zero_body, 0)

    def _hzero_body(b, carry):
        plsc.store_scatter(hist, [lanebase + b], zeros16i, mask=ones_m)
        return carry

    lax.fori_loop(0, NBUCKETS, _hzero_body, 0)

    def _hist_body(i, carry):
        fv = flat_v[pl.ds(i * 16, 16)]
        ch = lax.shift_right_logical(fv, SHIFT)
        valid = ch < NBUCKETS
        chc = jnp.minimum(ch, NBUCKETS - 1)
        cur = plsc.load_gather(hist, [lanebase + chc], mask=valid)
        plsc.store_scatter(hist, [lanebase + chc], cur + 1, mask=valid)
        return carry

    lax.fori_loop(0, PT // 16, _hist_body, 0)

    def _off_body(b, carry):
        bb = zeros16i + b
        cv = plsc.load_gather(hist, [lanebase + bb], mask=ones_m)
        inc = plsc.cumsum(cv)
        total = inc[15]
        plsc.store_scatter(tails, [lanebase + bb], carry + inc - cv,
                           mask=ones_m)
        lane0 = iota < 1
        plsc.store_scatter(starts, [bb], zeros16i + carry, mask=lane0)
        plsc.store_scatter(bcount, [bb], zeros16i + total, mask=lane0)
        return carry + ((total // WAVE) + 1) * WAVE

    lax.fori_loop(0, NBUCKETS, _off_body, 0)

    def _place_body(i, carry):
        fv = flat_v[pl.ds(i * 16, 16)]
        ch = lax.shift_right_logical(fv, SHIFT)
        valid = ch < NBUCKETS
        chc = jnp.minimum(ch, NBUCKETS - 1)
        dest = plsc.load_gather(tails, [lanebase + chc], mask=valid)
        plsc.store_scatter(sc_list, [dest], fv & (CHUNK - 1), mask=valid)
        plsc.store_scatter(sp_list, [dest], iota + (base + i * 16),
                           mask=valid)
        plsc.store_scatter(tails, [lanebase + chc], dest + 1, mask=valid)
        return carry

    lax.fori_loop(0, PT // 16, _place_body, 0)

    def _fill_body(b, carry):
        bb = zeros16i + b
        st = plsc.load_gather(starts, [bb])[0]
        cnt = plsc.load_gather(bcount, [bb])[0]
        t = st + cnt
        end = st + ((cnt + WAVE - 1) // WAVE) * WAVE
        dummy = zeros16i + DUMMY_CELL
        for k in range(WAVE // 16):
            offs = t + k * 16 + iota
            mfill = offs < end
            plsc.store_scatter(sc_list, [offs], dummy, mask=mfill)
            plsc.store_scatter(sp_list, [offs], zeros16i, mask=mfill)
        return carry

    lax.fori_loop(0, NBUCKETS, _fill_body, 0)

    zrows = zbuf.shape[0]

    def _pass_body(p, carry):
        chunk_id = c * PASSES + p
        lo = chunk_id * CHUNK
        bb = zeros16i + chunk_id
        st = pl.multiple_of(plsc.load_gather(starts, [bb])[0], WAVE)
        cnt = plsc.load_gather(bcount, [bb])[0]
        n_waves = (cnt + WAVE - 1) // WAVE

        def _clear_body(q, carry2):
            pltpu.sync_copy(
                zbuf, chunk.at[pl.ds(s * ROWS_PER_SUB + q * zrows, zrows)])
            return carry2

        lax.fori_loop(0, ROWS_PER_SUB // zrows, _clear_body, 0)
        plsc.subcore_barrier()

        def _wave_body(j, carry2):
            off = st + j * WAVE
            for k in range(WAVE // 16):
                cell_buf[pl.ds(k * 16, 16)] = sc_list[pl.ds(off + k * 16,
                                                            16)]
                pid_buf[pl.ds(k * 16, 16)] = sp_list[pl.ds(off + k * 16,
                                                           16)]
            pltpu.sync_copy(h_hbm.at[pid_buf], rows_v)
            pltpu.sync_copy(rows_v, chunk.at[cell_buf], add=True)
            return carry2

        lax.fori_loop(0, n_waves, _wave_body, 0)
        plsc.subcore_barrier()

        pltpu.sync_copy(
            chunk.at[pl.ds(s * ROWS_PER_SUB, ROWS_PER_SUB)],
            out_hbm.at[pl.ds(lo + s * ROWS_PER_SUB, ROWS_PER_SUB)])
        plsc.subcore_barrier()
        return carry

    lax.fori_loop(0, PASSES, _pass_body, 0)


def kernel(x, indices, W, b, gamma, beta):
    x = x.astype(jnp.float32)
    indices = indices.astype(jnp.int32)
    xi = jnp.pad(indices[:, 0], (0, P_PAD - N_POINTS),
                 constant_values=1 << 20)
    yi = jnp.pad(indices[:, 1], (0, P_PAD - N_POINTS))
    x_pad = jnp.pad(x, ((0, P_PAD - N_POINTS), (0, 0)))
    wt = W.astype(jnp.float32).T
    b2d = b.astype(jnp.float32).reshape(1, OUT_F)
    g2d = gamma.astype(jnp.float32).reshape(1, OUT_F)
    beta2d = beta.astype(jnp.float32).reshape(1, OUT_F)

    sx, sxx = pl.pallas_call(
        _stats_kernel,
        grid=(N_BLOCKS,),
        in_specs=[pl.BlockSpec((BK, IN_F), lambda i: (i, 0))],
        out_specs=[pl.BlockSpec((1, IN_F), lambda i: (0, 0)),
                   pl.BlockSpec((IN_F, IN_F), lambda i: (0, 0))],
        out_shape=[jax.ShapeDtypeStruct((1, IN_F), jnp.float32),
                   jax.ShapeDtypeStruct((IN_F, IN_F), jnp.float32)],
    )(x_pad)

    h = pl.pallas_call(
        _forward_kernel,
        grid=(N_BLOCKS,),
        in_specs=[pl.BlockSpec((BK, IN_F), lambda i: (i, 0)),
                  pl.BlockSpec((1, IN_F), lambda i: (0, 0)),
                  pl.BlockSpec((IN_F, IN_F), lambda i: (0, 0)),
                  pl.BlockSpec((IN_F, OUT_F), lambda i: (0, 0)),
                  pl.BlockSpec((1, OUT_F), lambda i: (0, 0)),
                  pl.BlockSpec((1, OUT_F), lambda i: (0, 0)),
                  pl.BlockSpec((1, OUT_F), lambda i: (0, 0))],
        out_specs=pl.BlockSpec((BK, OUT_F), lambda i: (i, 0)),
        out_shape=jax.ShapeDtypeStruct((P_PAD, OUT_F), jnp.float32),
    )(x_pad, sx, sxx, wt, b2d, g2d, beta2d)

    mesh = plsc.VectorSubcoreMesh(core_axis_name="c", subcore_axis_name="s")
    grid_flat = pl.kernel(
        _scatter_kernel,
        mesh=mesh,
        out_type=jax.ShapeDtypeStruct((N_PX * N_PY, OUT_F), jnp.float32),
        compiler_params=pltpu.CompilerParams(use_tc_tiling_on_sc=False,
                                             needs_layout_passes=False),
        scratch_types=[
            pltpu.VMEM((LIST_CAP,), jnp.int32),
            pltpu.VMEM((LIST_CAP,), jnp.int32),
            pltpu.VMEM((PT,), jnp.int32),
            pltpu.VMEM((16 * NBUCKETS,), jnp.int32),
            pltpu.VMEM((16 * NBUCKETS,), jnp.int32),
            pltpu.VMEM((NBUCKETS,), jnp.int32),
            pltpu.VMEM((NBUCKETS,), jnp.int32),
            pltpu.VMEM((WAVE,), jnp.int32),
            pltpu.VMEM((WAVE,), jnp.int32),
            pltpu.VMEM((WAVE, OUT_F), jnp.float32),
            pltpu.VMEM((256, OUT_F), jnp.float32),
            pltpu.VMEM_SHARED((CHUNK + 8, OUT_F), jnp.float32),
        ],
    )(xi, yi, h)

    return grid_flat.reshape(N_PX, N_PY, OUT_F)


# speedup vs baseline: 1.6349x; 1.6349x over previous
import jax
import jax.numpy as jnp
from jax import lax
from jax.experimental import pallas as pl
from jax.experimental.pallas import tpu as pltpu
from jax.experimental.pallas import tpu_sc as plsc

N_POINTS = 200000
N_PX = 512
N_PY = 512
IN_F = 6
OUT_F = 64
EPS = 1e-5

BK = 1024
P_PAD = 200704
N_BLOCKS = P_PAD // BK

NCORE = 2
NSUB = 16
PT = P_PAD // NSUB
CHUNK = 8192
SHIFT = 13
NBUCKETS = (N_PX * N_PY) // CHUNK
PASSES = NBUCKETS // NCORE
WAVE = 128
BIGWAVE = 512
LIST_CAP = PT + NBUCKETS * WAVE
ROWS_PER_SUB = CHUNK // NSUB
DUMMY_CELL = CHUNK
DUMMY_FLAG = 1 << 19


def _stats_kernel(x_ref, sx_ref, sxx_ref):
    i = pl.program_id(0)
    xb = x_ref[...]
    sx = jnp.sum(xb, axis=0, keepdims=True)
    sxx = lax.dot_general(xb, xb, (((0,), (0,)), ((), ())),
                          preferred_element_type=jnp.float32)

    @pl.when(i == 0)
    def _():
        sx_ref[...] = jnp.zeros_like(sx_ref)
        sxx_ref[...] = jnp.zeros_like(sxx_ref)

    sx_ref[...] += sx
    sxx_ref[...] += sxx


def _forward_kernel(x_ref, sx_ref, sxx_ref, wt_ref, b_ref, g_ref, beta_ref,
                    h_ref):
    mu = sx_ref[...] / N_POINTS
    m2 = sxx_ref[...] / N_POINTS
    wt = wt_ref[...]
    b = b_ref[...]
    mean = lax.dot_general(mu, wt, (((1,), (0,)), ((), ())),
                           preferred_element_type=jnp.float32) + b
    m2w = lax.dot_general(m2, wt, (((1,), (0,)), ((), ())),
                          preferred_element_type=jnp.float32)
    quad = jnp.sum(wt * m2w, axis=0, keepdims=True)
    var = quad + 2.0 * b * mean - b * b - mean * mean
    s = g_ref[...] * lax.rsqrt(var + EPS)
    b2 = (b - mean) * s + beta_ref[...]
    h = lax.dot_general(x_ref[...], wt, (((1,), (0,)), ((), ())),
                        preferred_element_type=jnp.float32)
    h_ref[...] = jnp.maximum(h * s + b2, 0.0)


def _scatter_kernel(xi_hbm, yi_hbm, h_hbm, out_hbm,
                    sp_list, flat_v, hist, tails, starts, bcount,
                    cell_buf, pid_buf, rows_v, cell_bbuf, pid_bbuf, rows_bv,
                    zbuf, chunk):
    c = lax.axis_index("c")
    s = lax.axis_index("s")
    base = s * PT
    iota = lax.iota(jnp.int32, 16)
    zeros16i = jnp.zeros((16,), jnp.int32)
    ones_m = iota < 16
    lanebase = iota * NBUCKETS

    pltpu.sync_copy(xi_hbm.at[pl.ds(base, PT)], flat_v)
    pltpu.sync_copy(yi_hbm.at[pl.ds(base, PT)], sp_list.at[pl.ds(0, PT)])

    def _flat_body(i, carry):
        xv = flat_v[pl.ds(i * 16, 16)]
        yv = sp_list[pl.ds(i * 16, 16)]
        flat_v[pl.ds(i * 16, 16)] = xv * N_PY + yv
        return carry

    lax.fori_loop(0, PT // 16, _flat_body, 0)

    zeros16f = jnp.zeros((16,), jnp.float32)

    def _zero_body(i, carry):
        for j in range(OUT_F // 16):
            zbuf[i, pl.ds(j * 16, 16)] = zeros16f
        return carry

    lax.fori_loop(0, zbuf.shape[0], _zero_body, 0)

    def _hzero_body(b, carry):
        plsc.store_scatter(hist, [lanebase + b], zeros16i, mask=ones_m)
        return carry

    lax.fori_loop(0, NBUCKETS, _hzero_body, 0)

    def _hist_body(i, carry):
        fv = flat_v[pl.ds(i * 16, 16)]
        ch = lax.shift_right_logical(fv, SHIFT)
        valid = ch < NBUCKETS
        chc = jnp.minimum(ch, NBUCKETS - 1)
        cur = plsc.load_gather(hist, [lanebase + chc], mask=valid)
        plsc.store_scatter(hist, [lanebase + chc], cur + 1, mask=valid)
        return carry

    lax.fori_loop(0, PT // 16, _hist_body, 0)

    def _off_body(b, carry):
        bb = zeros16i + b
        cv = plsc.load_gather(hist, [lanebase + bb], mask=ones_m)
        inc = plsc.cumsum(cv)
        total = inc[15]
        plsc.store_scatter(tails, [lanebase + bb], carry + inc - cv,
                           mask=ones_m)
        lane0 = iota < 1
        plsc.store_scatter(starts, [bb], zeros16i + carry, mask=lane0)
        plsc.store_scatter(bcount, [bb], zeros16i + total, mask=lane0)
        return carry + ((total // WAVE) + 1) * WAVE

    lax.fori_loop(0, NBUCKETS, _off_body, 0)

    def _place_body(i, carry):
        fv = flat_v[pl.ds(i * 16, 16)]
        ch = lax.shift_right_logical(fv, SHIFT)
        valid = ch < NBUCKETS
        chc = jnp.minimum(ch, NBUCKETS - 1)
        dest = plsc.load_gather(tails, [lanebase + chc], mask=valid)
        plsc.store_scatter(sp_list, [dest], iota + (base + i * 16),
                           mask=valid)
        plsc.store_scatter(tails, [lanebase + chc], dest + 1, mask=valid)
        return carry

    lax.fori_loop(0, PT // 16, _place_body, 0)

    dummyv = zeros16i + (base + DUMMY_FLAG)

    def _fill_body(b, carry):
        bb = zeros16i + b
        st = plsc.load_gather(starts, [bb])[0]
        cnt = plsc.load_gather(bcount, [bb])[0]
        t = st + cnt
        end = st + ((cnt + WAVE - 1) // WAVE) * WAVE
        for k in range(WAVE // 16):
            offs = t + k * 16 + iota
            mfill = offs < end
            plsc.store_scatter(sp_list, [offs], dummyv, mask=mfill)
        return carry

    lax.fori_loop(0, NBUCKETS, _fill_body, 0)

    zrows = zbuf.shape[0]

    def _stage(buf_cell, buf_pid, off, nvreg):
        for k in range(nvreg):
            pidv = sp_list[pl.ds(off + k * 16, 16)]
            isd = pidv >= DUMMY_FLAG
            rp = pidv & (DUMMY_FLAG - 1)
            buf_pid[pl.ds(k * 16, 16)] = rp
            fl = plsc.load_gather(flat_v, [rp - base])
            buf_cell[pl.ds(k * 16, 16)] = jnp.where(
                isd, DUMMY_CELL, fl & (CHUNK - 1))

    def _pass_body(p, carry):
        chunk_id = c * PASSES + p
        lo = chunk_id * CHUNK
        bb = zeros16i + chunk_id
        st = pl.multiple_of(plsc.load_gather(starts, [bb])[0], WAVE)
        cnt = plsc.load_gather(bcount, [bb])[0]
        n_waves = (cnt + WAVE - 1) // WAVE

        def _clear_body(q, carry2):
            pltpu.sync_copy(
                zbuf, chunk.at[pl.ds(s * ROWS_PER_SUB + q * zrows, zrows)])
            return carry2

        lax.fori_loop(0, ROWS_PER_SUB // zrows, _clear_body, 0)
        plsc.subcore_barrier()

        n_big = cnt // BIGWAVE

        def _bigwave_body(j, carry2):
            _stage(cell_bbuf, pid_bbuf, st + j * BIGWAVE, BIGWAVE // 16)
            pltpu.sync_copy(h_hbm.at[pid_bbuf], rows_bv)
            pltpu.sync_copy(rows_bv, chunk.at[cell_bbuf], add=True)
            return carry2

        lax.fori_loop(0, n_big, _bigwave_body, 0)

        def _wave_body(j, carry2):
            _stage(cell_buf, pid_buf, st + j * WAVE, WAVE // 16)
            pltpu.sync_copy(h_hbm.at[pid_buf], rows_v)
            pltpu.sync_copy(rows_v, chunk.at[cell_buf], add=True)
            return carry2

        lax.fori_loop(n_big * (BIGWAVE // WAVE), n_waves, _wave_body, 0)
        plsc.subcore_barrier()

        pltpu.sync_copy(
            chunk.at[pl.ds(s * ROWS_PER_SUB, ROWS_PER_SUB)],
            out_hbm.at[pl.ds(lo + s * ROWS_PER_SUB, ROWS_PER_SUB)])
        plsc.subcore_barrier()
        return carry

    lax.fori_loop(0, PASSES, _pass_body, 0)


def kernel(x, indices, W, b, gamma, beta):
    x = x.astype(jnp.float32)
    indices = indices.astype(jnp.int32)
    xi = jnp.pad(indices[:, 0], (0, P_PAD - N_POINTS),
                 constant_values=1 << 20)
    yi = jnp.pad(indices[:, 1], (0, P_PAD - N_POINTS))
    x_pad = jnp.pad(x, ((0, P_PAD - N_POINTS), (0, 0)))
    wt = W.astype(jnp.float32).T
    b2d = b.astype(jnp.float32).reshape(1, OUT_F)
    g2d = gamma.astype(jnp.float32).reshape(1, OUT_F)
    beta2d = beta.astype(jnp.float32).reshape(1, OUT_F)

    sx, sxx = pl.pallas_call(
        _stats_kernel,
        grid=(N_BLOCKS,),
        in_specs=[pl.BlockSpec((BK, IN_F), lambda i: (i, 0))],
        out_specs=[pl.BlockSpec((1, IN_F), lambda i: (0, 0)),
                   pl.BlockSpec((IN_F, IN_F), lambda i: (0, 0))],
        out_shape=[jax.ShapeDtypeStruct((1, IN_F), jnp.float32),
                   jax.ShapeDtypeStruct((IN_F, IN_F), jnp.float32)],
    )(x_pad)

    h = pl.pallas_call(
        _forward_kernel,
        grid=(N_BLOCKS,),
        in_specs=[pl.BlockSpec((BK, IN_F), lambda i: (i, 0)),
                  pl.BlockSpec((1, IN_F), lambda i: (0, 0)),
                  pl.BlockSpec((IN_F, IN_F), lambda i: (0, 0)),
                  pl.BlockSpec((IN_F, OUT_F), lambda i: (0, 0)),
                  pl.BlockSpec((1, OUT_F), lambda i: (0, 0)),
                  pl.BlockSpec((1, OUT_F), lambda i: (0, 0)),
                  pl.BlockSpec((1, OUT_F), lambda i: (0, 0))],
        out_specs=pl.BlockSpec((BK, OUT_F), lambda i: (i, 0)),
        out_shape=jax.ShapeDtypeStruct((P_PAD, OUT_F), jnp.float32),
    )(x_pad, sx, sxx, wt, b2d, g2d, beta2d)

    mesh = plsc.VectorSubcoreMesh(core_axis_name="c", subcore_axis_name="s")
    grid_flat = pl.kernel(
        _scatter_kernel,
        mesh=mesh,
        out_type=jax.ShapeDtypeStruct((N_PX * N_PY, OUT_F), jnp.float32),
        compiler_params=pltpu.CompilerParams(use_tc_tiling_on_sc=False,
                                             needs_layout_passes=False),
        scratch_types=[
            pltpu.VMEM((LIST_CAP,), jnp.int32),
            pltpu.VMEM((PT,), jnp.int32),
            pltpu.VMEM((16 * NBUCKETS,), jnp.int32),
            pltpu.VMEM((16 * NBUCKETS,), jnp.int32),
            pltpu.VMEM((NBUCKETS,), jnp.int32),
            pltpu.VMEM((NBUCKETS,), jnp.int32),
            pltpu.VMEM((WAVE,), jnp.int32),
            pltpu.VMEM((WAVE,), jnp.int32),
            pltpu.VMEM((WAVE, OUT_F), jnp.float32),
            pltpu.VMEM((BIGWAVE,), jnp.int32),
            pltpu.VMEM((BIGWAVE,), jnp.int32),
            pltpu.VMEM((BIGWAVE, OUT_F), jnp.float32),
            pltpu.VMEM((128, OUT_F), jnp.float32),
            pltpu.VMEM_SHARED((CHUNK + 8, OUT_F), jnp.float32),
        ],
    )(xi, yi, h)

    return grid_flat.reshape(N_PX, N_PY, OUT_F)


# speedup vs baseline: 2.1233x; 1.2987x over previous
import jax
import jax.numpy as jnp
from jax import lax
from jax.experimental import pallas as pl
from jax.experimental.pallas import tpu as pltpu
from jax.experimental.pallas import tpu_sc as plsc

N_POINTS = 200000
N_PX = 512
N_PY = 512
IN_F = 6
OUT_F = 64
EPS = 1e-5

BK = 4096
P_PAD = 200704
N_BLOCKS = P_PAD // BK

NCORE = 2
NSUB = 16
PT = P_PAD // NSUB
CHUNK = 8192
SHIFT = 13
NBUCKETS = (N_PX * N_PY) // CHUNK
PASSES = NBUCKETS // NCORE
WAVE = 128
BIGWAVE = 512
LIST_CAP = PT + NBUCKETS * WAVE
ROWS_PER_SUB = CHUNK // NSUB
DUMMY_CELL = CHUNK
DUMMY_FLAG = 1 << 19


def _stats_kernel(x_ref, sx_ref, sxx_ref):
    i = pl.program_id(0)
    rows = lax.broadcasted_iota(jnp.int32, (BK, 1), 0) + i * BK
    xb = jnp.where(rows < N_POINTS, x_ref[...], 0.0)
    sx = jnp.sum(xb, axis=0, keepdims=True)
    sxx = lax.dot_general(xb, xb, (((0,), (0,)), ((), ())),
                          preferred_element_type=jnp.float32)

    @pl.when(i == 0)
    def _():
        sx_ref[...] = jnp.zeros_like(sx_ref)
        sxx_ref[...] = jnp.zeros_like(sxx_ref)

    sx_ref[...] += sx
    sxx_ref[...] += sxx


def _forward_kernel(x_ref, sx_ref, sxx_ref, wt_ref, b_ref, g_ref, beta_ref,
                    h_ref):
    mu = sx_ref[...] / N_POINTS
    m2 = sxx_ref[...] / N_POINTS
    wt = wt_ref[...]
    b = b_ref[...]
    mean = lax.dot_general(mu, wt, (((1,), (0,)), ((), ())),
                           preferred_element_type=jnp.float32) + b
    m2w = lax.dot_general(m2, wt, (((1,), (0,)), ((), ())),
                          preferred_element_type=jnp.float32)
    quad = jnp.sum(wt * m2w, axis=0, keepdims=True)
    var = quad + 2.0 * b * mean - b * b - mean * mean
    s = g_ref[...] * lax.rsqrt(var + EPS)
    b2 = (b - mean) * s + beta_ref[...]
    h = lax.dot_general(x_ref[...], wt, (((1,), (0,)), ((), ())),
                        preferred_element_type=jnp.float32)
    h_ref[...] = jnp.maximum(h * s + b2, 0.0)


def _scatter_kernel(xi_hbm, yi_hbm, h_hbm, out_hbm,
                    sp_list, flat_v, hist, tails, starts, bcount,
                    cell_buf, pid_buf, rows_v, cell_bbuf, pid_bbuf, rows_bv,
                    zbuf, chunk):
    c = lax.axis_index("c")
    s = lax.axis_index("s")
    base = s * PT
    iota = lax.iota(jnp.int32, 16)
    zeros16i = jnp.zeros((16,), jnp.int32)
    ones_m = iota < 16
    lanebase = iota * NBUCKETS

    pltpu.sync_copy(xi_hbm.at[pl.ds(base, PT)], flat_v)
    pltpu.sync_copy(yi_hbm.at[pl.ds(base, PT)], sp_list.at[pl.ds(0, PT)])

    def _flat_body(i, carry):
        xv = flat_v[pl.ds(i * 16, 16)]
        yv = sp_list[pl.ds(i * 16, 16)]
        flat_v[pl.ds(i * 16, 16)] = xv * N_PY + yv
        return carry

    lax.fori_loop(0, PT // 16, _flat_body, 0)

    zeros16f = jnp.zeros((16,), jnp.float32)

    def _zero_body(i, carry):
        for j in range(OUT_F // 16):
            zbuf[i, pl.ds(j * 16, 16)] = zeros16f
        return carry

    lax.fori_loop(0, zbuf.shape[0], _zero_body, 0)

    def _hzero_body(b, carry):
        plsc.store_scatter(hist, [lanebase + b], zeros16i, mask=ones_m)
        return carry

    lax.fori_loop(0, NBUCKETS, _hzero_body, 0)

    def _hist_body(i, carry):
        fv = flat_v[pl.ds(i * 16, 16)]
        ch = lax.shift_right_logical(fv, SHIFT)
        valid = ch < NBUCKETS
        chc = jnp.minimum(ch, NBUCKETS - 1)
        cur = plsc.load_gather(hist, [lanebase + chc], mask=valid)
        plsc.store_scatter(hist, [lanebase + chc], cur + 1, mask=valid)
        return carry

    lax.fori_loop(0, PT // 16, _hist_body, 0)

    def _off_body(b, carry):
        bb = zeros16i + b
        cv = plsc.load_gather(hist, [lanebase + bb], mask=ones_m)
        inc = plsc.cumsum(cv)
        total = inc[15]
        plsc.store_scatter(tails, [lanebase + bb], carry + inc - cv,
                           mask=ones_m)
        lane0 = iota < 1
        plsc.store_scatter(starts, [bb], zeros16i + carry, mask=lane0)
        plsc.store_scatter(bcount, [bb], zeros16i + total, mask=lane0)
        return carry + ((total // WAVE) + 1) * WAVE

    lax.fori_loop(0, NBUCKETS, _off_body, 0)

    def _place_body(i, carry):
        fv = flat_v[pl.ds(i * 16, 16)]
        ch = lax.shift_right_logical(fv, SHIFT)
        valid = ch < NBUCKETS
        chc = jnp.minimum(ch, NBUCKETS - 1)
        dest = plsc.load_gather(tails, [lanebase + chc], mask=valid)
        plsc.store_scatter(sp_list, [dest], iota + (base + i * 16),
                           mask=valid)
        plsc.store_scatter(tails, [lanebase + chc], dest + 1, mask=valid)
        return carry

    lax.fori_loop(0, PT // 16, _place_body, 0)

    dummyv = zeros16i + (base + DUMMY_FLAG)

    def _fill_body(b, carry):
        bb = zeros16i + b
        st = plsc.load_gather(starts, [bb])[0]
        cnt = plsc.load_gather(bcount, [bb])[0]
        t = st + cnt
        end = st + ((cnt + WAVE - 1) // WAVE) * WAVE
        for k in range(WAVE // 16):
            offs = t + k * 16 + iota
            mfill = offs < end
            plsc.store_scatter(sp_list, [offs], dummyv, mask=mfill)
        return carry

    lax.fori_loop(0, NBUCKETS, _fill_body, 0)

    zrows = zbuf.shape[0]

    def _stage(buf_cell, buf_pid, off, nvreg):
        for k in range(nvreg):
            pidv = sp_list[pl.ds(off + k * 16, 16)]
            isd = pidv >= DUMMY_FLAG
            rp = pidv & (DUMMY_FLAG - 1)
            buf_pid[pl.ds(k * 16, 16)] = rp
            fl = plsc.load_gather(flat_v, [rp - base])
            buf_cell[pl.ds(k * 16, 16)] = jnp.where(
                isd, DUMMY_CELL, fl & (CHUNK - 1))

    def _pass_body(p, carry):
        chunk_id = c * PASSES + p
        lo = chunk_id * CHUNK
        bb = zeros16i + chunk_id
        st = pl.multiple_of(plsc.load_gather(starts, [bb])[0], WAVE)
        cnt = plsc.load_gather(bcount, [bb])[0]
        n_waves = (cnt + WAVE - 1) // WAVE

        def _clear_body(q, carry2):
            pltpu.sync_copy(
                zbuf, chunk.at[pl.ds(s * ROWS_PER_SUB + q * zrows, zrows)])
            return carry2

        lax.fori_loop(0, ROWS_PER_SUB // zrows, _clear_body, 0)
        plsc.subcore_barrier()

        n_big = cnt // BIGWAVE

        def _bigwave_body(j, carry2):
            _stage(cell_bbuf, pid_bbuf, st + j * BIGWAVE, BIGWAVE // 16)
            pltpu.sync_copy(h_hbm.at[pid_bbuf], rows_bv)
            pltpu.sync_copy(rows_bv, chunk.at[cell_bbuf], add=True)
            return carry2

        lax.fori_loop(0, n_big, _bigwave_body, 0)

        def _wave_body(j, carry2):
            _stage(cell_buf, pid_buf, st + j * WAVE, WAVE // 16)
            pltpu.sync_copy(h_hbm.at[pid_buf], rows_v)
            pltpu.sync_copy(rows_v, chunk.at[cell_buf], add=True)
            return carry2

        lax.fori_loop(n_big * (BIGWAVE // WAVE), n_waves, _wave_body, 0)
        plsc.subcore_barrier()

        pltpu.sync_copy(
            chunk.at[pl.ds(s * ROWS_PER_SUB, ROWS_PER_SUB)],
            out_hbm.at[pl.ds(lo + s * ROWS_PER_SUB, ROWS_PER_SUB)])
        plsc.subcore_barrier()
        return carry

    lax.fori_loop(0, PASSES, _pass_body, 0)


def kernel(x, indices, W, b, gamma, beta):
    x = x.astype(jnp.float32)
    indices = indices.astype(jnp.int32)
    xi = jnp.pad(indices[:, 0], (0, P_PAD - N_POINTS),
                 constant_values=1 << 20)
    yi = jnp.pad(indices[:, 1], (0, P_PAD - N_POINTS))
    wt = W.astype(jnp.float32).T
    b2d = b.astype(jnp.float32).reshape(1, OUT_F)
    g2d = gamma.astype(jnp.float32).reshape(1, OUT_F)
    beta2d = beta.astype(jnp.float32).reshape(1, OUT_F)

    sx, sxx = pl.pallas_call(
        _stats_kernel,
        grid=(N_BLOCKS,),
        in_specs=[pl.BlockSpec((BK, IN_F), lambda i: (i, 0))],
        out_specs=[pl.BlockSpec((1, IN_F), lambda i: (0, 0)),
                   pl.BlockSpec((IN_F, IN_F), lambda i: (0, 0))],
        out_shape=[jax.ShapeDtypeStruct((1, IN_F), jnp.float32),
                   jax.ShapeDtypeStruct((IN_F, IN_F), jnp.float32)],
    )(x)

    h = pl.pallas_call(
        _forward_kernel,
        grid=(N_BLOCKS,),
        in_specs=[pl.BlockSpec((BK, IN_F), lambda i: (i, 0)),
                  pl.BlockSpec((1, IN_F), lambda i: (0, 0)),
                  pl.BlockSpec((IN_F, IN_F), lambda i: (0, 0)),
                  pl.BlockSpec((IN_F, OUT_F), lambda i: (0, 0)),
                  pl.BlockSpec((1, OUT_F), lambda i: (0, 0)),
                  pl.BlockSpec((1, OUT_F), lambda i: (0, 0)),
                  pl.BlockSpec((1, OUT_F), lambda i: (0, 0))],
        out_specs=pl.BlockSpec((BK, OUT_F), lambda i: (i, 0)),
        out_shape=jax.ShapeDtypeStruct((P_PAD, OUT_F), jnp.float32),
    )(x, sx, sxx, wt, b2d, g2d, beta2d)

    mesh = plsc.VectorSubcoreMesh(core_axis_name="c", subcore_axis_name="s")
    grid_flat = pl.kernel(
        _scatter_kernel,
        mesh=mesh,
        out_type=jax.ShapeDtypeStruct((N_PX * N_PY, OUT_F), jnp.float32),
        compiler_params=pltpu.CompilerParams(use_tc_tiling_on_sc=False,
                                             needs_layout_passes=False),
        scratch_types=[
            pltpu.VMEM((LIST_CAP,), jnp.int32),
            pltpu.VMEM((PT,), jnp.int32),
            pltpu.VMEM((16 * NBUCKETS,), jnp.int32),
            pltpu.VMEM((16 * NBUCKETS,), jnp.int32),
            pltpu.VMEM((NBUCKETS,), jnp.int32),
            pltpu.VMEM((NBUCKETS,), jnp.int32),
            pltpu.VMEM((WAVE,), jnp.int32),
            pltpu.VMEM((WAVE,), jnp.int32),
            pltpu.VMEM((WAVE, OUT_F), jnp.float32),
            pltpu.VMEM((BIGWAVE,), jnp.int32),
            pltpu.VMEM((BIGWAVE,), jnp.int32),
            pltpu.VMEM((BIGWAVE, OUT_F), jnp.float32),
            pltpu.VMEM((128, OUT_F), jnp.float32),
            pltpu.VMEM_SHARED((CHUNK + 8, OUT_F), jnp.float32),
        ],
    )(xi, yi, h)

    return grid_flat.reshape(N_PX, N_PY, OUT_F)


# speedup vs baseline: 2.1571x; 1.0159x over previous
import jax
import jax.numpy as jnp
from jax import lax
from jax.experimental import pallas as pl
from jax.experimental.pallas import tpu as pltpu
from jax.experimental.pallas import tpu_sc as plsc

N_POINTS = 200000
N_PX = 512
N_PY = 512
IN_F = 6
OUT_F = 64
EPS = 1e-5

BK = 4096
P_PAD = 200704
N_BLOCKS = P_PAD // BK

NCORE = 2
NSUB = 16
PT = P_PAD // NSUB
CHUNK = 8192
SHIFT = 13
NBUCKETS = (N_PX * N_PY) // CHUNK
PASSES = NBUCKETS // NCORE
WAVE = 128
BIGWAVE = 256
LIST_CAP = PT + NBUCKETS * WAVE
ROWS_PER_SUB = CHUNK // NSUB
DUMMY_CELL = CHUNK
DUMMY_FLAG = 1 << 19


def _stats_kernel(x_ref, sx_ref, sxx_ref):
    i = pl.program_id(0)
    rows = lax.broadcasted_iota(jnp.int32, (BK, 1), 0) + i * BK
    xb = jnp.where(rows < N_POINTS, x_ref[...], 0.0)
    sx = jnp.sum(xb, axis=0, keepdims=True)
    sxx = lax.dot_general(xb, xb, (((0,), (0,)), ((), ())),
                          preferred_element_type=jnp.float32)

    @pl.when(i == 0)
    def _():
        sx_ref[...] = jnp.zeros_like(sx_ref)
        sxx_ref[...] = jnp.zeros_like(sxx_ref)

    sx_ref[...] += sx
    sxx_ref[...] += sxx


def _forward_kernel(x_ref, sx_ref, sxx_ref, wt_ref, b_ref, g_ref, beta_ref,
                    h_ref):
    mu = sx_ref[...] / N_POINTS
    m2 = sxx_ref[...] / N_POINTS
    wt = wt_ref[...]
    b = b_ref[...]
    mean = lax.dot_general(mu, wt, (((1,), (0,)), ((), ())),
                           preferred_element_type=jnp.float32) + b
    m2w = lax.dot_general(m2, wt, (((1,), (0,)), ((), ())),
                          preferred_element_type=jnp.float32)
    quad = jnp.sum(wt * m2w, axis=0, keepdims=True)
    var = quad + 2.0 * b * mean - b * b - mean * mean
    s = g_ref[...] * lax.rsqrt(var + EPS)
    b2 = (b - mean) * s + beta_ref[...]
    h = lax.dot_general(x_ref[...], wt, (((1,), (0,)), ((), ())),
                        preferred_element_type=jnp.float32)
    h_ref[...] = jnp.maximum(h * s + b2, 0.0)


def _scatter_kernel(xi_hbm, yi_hbm, h_hbm, out_hbm,
                    sp_list, flat_v, hist, tails, starts, bcount,
                    cell_buf, pid_buf, rows_v, cell_bb0, pid_bb0, rows_b0,
                    cell_bb1, pid_bb1, rows_b1, gsem0, gsem1, asem0, asem1,
                    zbuf, chunk):
    c = lax.axis_index("c")
    s = lax.axis_index("s")
    base = s * PT
    iota = lax.iota(jnp.int32, 16)
    zeros16i = jnp.zeros((16,), jnp.int32)
    ones_m = iota < 16
    lanebase = iota * NBUCKETS

    pltpu.sync_copy(xi_hbm.at[pl.ds(base, PT)], flat_v)
    pltpu.sync_copy(yi_hbm.at[pl.ds(base, PT)], sp_list.at[pl.ds(0, PT)])

    def _flat_body(i, carry):
        xv = flat_v[pl.ds(i * 16, 16)]
        yv = sp_list[pl.ds(i * 16, 16)]
        flat_v[pl.ds(i * 16, 16)] = xv * N_PY + yv
        return carry

    lax.fori_loop(0, PT // 16, _flat_body, 0)

    zeros16f = jnp.zeros((16,), jnp.float32)

    def _zero_body(i, carry):
        for j in range(OUT_F // 16):
            zbuf[i, pl.ds(j * 16, 16)] = zeros16f
        return carry

    lax.fori_loop(0, zbuf.shape[0], _zero_body, 0)

    def _hzero_body(b, carry):
        plsc.store_scatter(hist, [lanebase + b], zeros16i, mask=ones_m)
        return carry

    lax.fori_loop(0, NBUCKETS, _hzero_body, 0)

    def _hist_body(i, carry):
        fv = flat_v[pl.ds(i * 16, 16)]
        ch = lax.shift_right_logical(fv, SHIFT)
        valid = ch < NBUCKETS
        chc = jnp.minimum(ch, NBUCKETS - 1)
        cur = plsc.load_gather(hist, [lanebase + chc], mask=valid)
        plsc.store_scatter(hist, [lanebase + chc], cur + 1, mask=valid)
        return carry

    lax.fori_loop(0, PT // 16, _hist_body, 0)

    def _off_body(b, carry):
        bb = zeros16i + b
        cv = plsc.load_gather(hist, [lanebase + bb], mask=ones_m)
        inc = plsc.cumsum(cv)
        total = inc[15]
        plsc.store_scatter(tails, [lanebase + bb], carry + inc - cv,
                           mask=ones_m)
        lane0 = iota < 1
        plsc.store_scatter(starts, [bb], zeros16i + carry, mask=lane0)
        plsc.store_scatter(bcount, [bb], zeros16i + total, mask=lane0)
        return carry + ((total // WAVE) + 1) * WAVE

    lax.fori_loop(0, NBUCKETS, _off_body, 0)

    def _place_body(i, carry):
        fv = flat_v[pl.ds(i * 16, 16)]
        ch = lax.shift_right_logical(fv, SHIFT)
        valid = ch < NBUCKETS
        chc = jnp.minimum(ch, NBUCKETS - 1)
        dest = plsc.load_gather(tails, [lanebase + chc], mask=valid)
        plsc.store_scatter(sp_list, [dest], iota + (base + i * 16),
                           mask=valid)
        plsc.store_scatter(tails, [lanebase + chc], dest + 1, mask=valid)
        return carry

    lax.fori_loop(0, PT // 16, _place_body, 0)

    dummyv = zeros16i + (base + DUMMY_FLAG)

    def _fill_body(b, carry):
        bb = zeros16i + b
        st = plsc.load_gather(starts, [bb])[0]
        cnt = plsc.load_gather(bcount, [bb])[0]
        t = st + cnt
        end = st + ((cnt + WAVE - 1) // WAVE) * WAVE
        for k in range(WAVE // 16):
            offs = t + k * 16 + iota
            mfill = offs < end
            plsc.store_scatter(sp_list, [offs], dummyv, mask=mfill)
        return carry

    lax.fori_loop(0, NBUCKETS, _fill_body, 0)

    zrows = zbuf.shape[0]

    def _stage(buf_cell, buf_pid, off, nvreg):
        for k in range(nvreg):
            pidv = sp_list[pl.ds(off + k * 16, 16)]
            isd = pidv >= DUMMY_FLAG
            rp = pidv & (DUMMY_FLAG - 1)
            buf_pid[pl.ds(k * 16, 16)] = rp
            fl = plsc.load_gather(flat_v, [rp - base])
            buf_cell[pl.ds(k * 16, 16)] = jnp.where(
                isd, DUMMY_CELL, fl & (CHUNK - 1))

    def _pass_body(p, carry):
        chunk_id = c * PASSES + p
        lo = chunk_id * CHUNK
        bb = zeros16i + chunk_id
        st = pl.multiple_of(plsc.load_gather(starts, [bb])[0], WAVE)
        cnt = plsc.load_gather(bcount, [bb])[0]
        n_waves = (cnt + WAVE - 1) // WAVE

        def _clear_body(q, carry2):
            pltpu.sync_copy(
                zbuf, chunk.at[pl.ds(s * ROWS_PER_SUB + q * zrows, zrows)])
            return carry2

        lax.fori_loop(0, ROWS_PER_SUB // zrows, _clear_body, 0)
        plsc.subcore_barrier()

        n_big = cnt // BIGWAVE
        bb_cell = (cell_bb0, cell_bb1)
        bb_pid = (pid_bb0, pid_bb1)
        bb_rows = (rows_b0, rows_b1)
        bb_gsem = (gsem0, gsem1)
        bb_asem = (asem0, asem1)

        def _bigpair(jj, carry2):
            for par in range(2):
                j = jj * 2 + par

                @pl.when(j < n_big)
                def _(par=par, j=j):
                    @pl.when(j >= 2)
                    def _():
                        pltpu.make_async_copy(
                            bb_rows[par], chunk.at[bb_cell[par]],
                            bb_asem[par]).wait()

                    _stage(bb_cell[par], bb_pid[par], st + j * BIGWAVE,
                           BIGWAVE // 16)
                    pltpu.async_copy(h_hbm.at[bb_pid[par]], bb_rows[par],
                                     bb_gsem[par])
            for par in range(2):
                j = jj * 2 + par

                @pl.when(j < n_big)
                def _(par=par, j=j):
                    pltpu.make_async_copy(h_hbm.at[bb_pid[par]],
                                          bb_rows[par], bb_gsem[par]).wait()
                    pltpu.async_copy(bb_rows[par], chunk.at[bb_cell[par]],
                                     bb_asem[par], add=True)
            return carry2

        lax.fori_loop(0, (n_big + 1) // 2, _bigpair, 0)
        for par in range(2):

            @pl.when(n_big > par)
            def _(par=par):
                pltpu.make_async_copy(bb_rows[par], chunk.at[bb_cell[par]],
                                      bb_asem[par]).wait()

        def _wave_body(j, carry2):
            _stage(cell_buf, pid_buf, st + j * WAVE, WAVE // 16)
            pltpu.sync_copy(h_hbm.at[pid_buf], rows_v)
            pltpu.sync_copy(rows_v, chunk.at[cell_buf], add=True)
            return carry2

        lax.fori_loop(n_big * (BIGWAVE // WAVE), n_waves, _wave_body, 0)
        plsc.subcore_barrier()

        pltpu.sync_copy(
            chunk.at[pl.ds(s * ROWS_PER_SUB, ROWS_PER_SUB)],
            out_hbm.at[pl.ds(lo + s * ROWS_PER_SUB, ROWS_PER_SUB)])
        plsc.subcore_barrier()
        return carry

    lax.fori_loop(0, PASSES, _pass_body, 0)


def kernel(x, indices, W, b, gamma, beta):
    x = x.astype(jnp.float32)
    indices = indices.astype(jnp.int32)
    xi = jnp.pad(indices[:, 0], (0, P_PAD - N_POINTS),
                 constant_values=1 << 20)
    yi = jnp.pad(indices[:, 1], (0, P_PAD - N_POINTS))
    wt = W.astype(jnp.float32).T
    b2d = b.astype(jnp.float32).reshape(1, OUT_F)
    g2d = gamma.astype(jnp.float32).reshape(1, OUT_F)
    beta2d = beta.astype(jnp.float32).reshape(1, OUT_F)

    sx, sxx = pl.pallas_call(
        _stats_kernel,
        grid=(N_BLOCKS,),
        in_specs=[pl.BlockSpec((BK, IN_F), lambda i: (i, 0))],
        out_specs=[pl.BlockSpec((1, IN_F), lambda i: (0, 0)),
                   pl.BlockSpec((IN_F, IN_F), lambda i: (0, 0))],
        out_shape=[jax.ShapeDtypeStruct((1, IN_F), jnp.float32),
                   jax.ShapeDtypeStruct((IN_F, IN_F), jnp.float32)],
    )(x)

    h = pl.pallas_call(
        _forward_kernel,
        grid=(N_BLOCKS,),
        in_specs=[pl.BlockSpec((BK, IN_F), lambda i: (i, 0)),
                  pl.BlockSpec((1, IN_F), lambda i: (0, 0)),
                  pl.BlockSpec((IN_F, IN_F), lambda i: (0, 0)),
                  pl.BlockSpec((IN_F, OUT_F), lambda i: (0, 0)),
                  pl.BlockSpec((1, OUT_F), lambda i: (0, 0)),
                  pl.BlockSpec((1, OUT_F), lambda i: (0, 0)),
                  pl.BlockSpec((1, OUT_F), lambda i: (0, 0))],
        out_specs=pl.BlockSpec((BK, OUT_F), lambda i: (i, 0)),
        out_shape=jax.ShapeDtypeStruct((P_PAD, OUT_F), jnp.float32),
    )(x, sx, sxx, wt, b2d, g2d, beta2d)

    mesh = plsc.VectorSubcoreMesh(core_axis_name="c", subcore_axis_name="s")
    grid_flat = pl.kernel(
        _scatter_kernel,
        mesh=mesh,
        out_type=jax.ShapeDtypeStruct((N_PX * N_PY, OUT_F), jnp.float32),
        compiler_params=pltpu.CompilerParams(use_tc_tiling_on_sc=False,
                                             needs_layout_passes=False),
        scratch_types=[
            pltpu.VMEM((LIST_CAP,), jnp.int32),
            pltpu.VMEM((PT,), jnp.int32),
            pltpu.VMEM((16 * NBUCKETS,), jnp.int32),
            pltpu.VMEM((16 * NBUCKETS,), jnp.int32),
            pltpu.VMEM((NBUCKETS,), jnp.int32),
            pltpu.VMEM((NBUCKETS,), jnp.int32),
            pltpu.VMEM((WAVE,), jnp.int32),
            pltpu.VMEM((WAVE,), jnp.int32),
            pltpu.VMEM((WAVE, OUT_F), jnp.float32),
            pltpu.VMEM((BIGWAVE,), jnp.int32),
            pltpu.VMEM((BIGWAVE,), jnp.int32),
            pltpu.VMEM((BIGWAVE, OUT_F), jnp.float32),
            pltpu.VMEM((BIGWAVE,), jnp.int32),
            pltpu.VMEM((BIGWAVE,), jnp.int32),
            pltpu.VMEM((BIGWAVE, OUT_F), jnp.float32),
            pltpu.SemaphoreType.DMA,
            pltpu.SemaphoreType.DMA,
            pltpu.SemaphoreType.DMA,
            pltpu.SemaphoreType.DMA,
            pltpu.VMEM((128, OUT_F), jnp.float32),
            pltpu.VMEM_SHARED((CHUNK + 8, OUT_F), jnp.float32),
        ],
    )(xi, yi, h)

    return grid_flat.reshape(N_PX, N_PY, OUT_F)


# speedup vs baseline: 2.1648x; 1.0036x over previous
import jax
import jax.numpy as jnp
from jax import lax
from jax.experimental import pallas as pl
from jax.experimental.pallas import tpu as pltpu
from jax.experimental.pallas import tpu_sc as plsc

N_POINTS = 200000
N_PX = 512
N_PY = 512
IN_F = 6
OUT_F = 64
EPS = 1e-5

P_PAD = 200704
PACKW = 64 * IN_F
XROWS = P_PAD * IN_F // PACKW
FBK = 448
N_FBLOCKS = XROWS // FBK

NCORE = 2
NSUB = 16
PT = P_PAD // NSUB
CHUNK = 8192
SHIFT = 13
NBUCKETS = (N_PX * N_PY) // CHUNK
PASSES = NBUCKETS // NCORE
WAVE = 128
BIGWAVE = 256
LIST_CAP = PT + NBUCKETS * WAVE
ROWS_PER_SUB = CHUNK // NSUB
DUMMY_CELL = CHUNK
DUMMY_FLAG = 1 << 19


def _stats_kernel(xc_ref, sx_ref, sxx_ref):
    xc = xc_ref[...]
    a = lax.dot_general(xc, xc, (((0,), (0,)), ((), ())),
                        preferred_element_type=jnp.float32)
    r = lax.broadcasted_iota(jnp.int32, (PACKW, PACKW), 0)
    cmat = lax.broadcasted_iota(jnp.int32, (PACKW, PACKW), 1)
    bd = jnp.where(r // IN_F == cmat // IN_F, 1.0, 0.0)
    m = jnp.where(lax.broadcasted_iota(jnp.int32, (PACKW, IN_F), 0) % IN_F
                  == lax.broadcasted_iota(jnp.int32, (PACKW, IN_F), 1),
                  1.0, 0.0)
    t1 = lax.dot_general(m, a * bd, (((0,), (0,)), ((), ())),
                         preferred_element_type=jnp.float32)
    sxx_ref[...] = lax.dot_general(t1, m, (((1,), (0,)), ((), ())),
                                   preferred_element_type=jnp.float32)
    xm = lax.dot_general(xc, m, (((1,), (0,)), ((), ())),
                         preferred_element_type=jnp.float32)
    sx_ref[...] = jnp.sum(xm, axis=0, keepdims=True)


def _fold_kernel(sx_ref, sxx_ref, wt_ref, b_ref, g_ref, beta_ref,
                 w2_ref, b2_ref):
    mu = sx_ref[...] / N_POINTS
    m2 = sxx_ref[...] / N_POINTS
    wt = wt_ref[...]
    b = b_ref[...]
    mean = lax.dot_general(mu, wt, (((1,), (0,)), ((), ())),
                           preferred_element_type=jnp.float32) + b
    m2w = lax.dot_general(m2, wt, (((1,), (0,)), ((), ())),
                          preferred_element_type=jnp.float32)
    quad = jnp.sum(wt * m2w, axis=0, keepdims=True)
    var = quad + 2.0 * b * mean - b * b - mean * mean
    sc = g_ref[...] * lax.rsqrt(var + EPS)
    w2_ref[...] = wt * sc
    b2_ref[...] = (b - mean) * sc + beta_ref[...]


def _forward_kernel(xc_ref, w384_ref, b4096_ref, h_ref):
    h = lax.dot_general(xc_ref[...], w384_ref[...], (((1,), (0,)), ((), ())),
                        preferred_element_type=jnp.float32)
    h_ref[...] = jnp.maximum(h + b4096_ref[...], 0.0)


def _scatter_kernel(xi_hbm, yi_hbm, h_hbm, out_hbm,
                    sp_list, flat_v, hist, tails, starts, bcount,
                    cell_buf, pid_buf, rows_v, cell_bb0, pid_bb0, rows_b0,
                    cell_bb1, pid_bb1, rows_b1, gsem0, gsem1, asem0, asem1,
                    zbuf, chunk):
    c = lax.axis_index("c")
    s = lax.axis_index("s")
    base = s * PT
    iota = lax.iota(jnp.int32, 16)
    zeros16i = jnp.zeros((16,), jnp.int32)
    ones_m = iota < 16
    lanebase = iota * NBUCKETS

    pltpu.sync_copy(xi_hbm.at[pl.ds(base, PT)], flat_v)
    pltpu.sync_copy(yi_hbm.at[pl.ds(base, PT)], sp_list.at[pl.ds(0, PT)])

    def _flat_body(i, carry):
        xv = flat_v[pl.ds(i * 16, 16)]
        yv = sp_list[pl.ds(i * 16, 16)]
        flat_v[pl.ds(i * 16, 16)] = xv * N_PY + yv
        return carry

    lax.fori_loop(0, PT // 16, _flat_body, 0)

    zeros16f = jnp.zeros((16,), jnp.float32)

    def _zero_body(i, carry):
        for j in range(OUT_F // 16):
            zbuf[i, pl.ds(j * 16, 16)] = zeros16f
        return carry

    lax.fori_loop(0, zbuf.shape[0], _zero_body, 0)

    def _hzero_body(b, carry):
        plsc.store_scatter(hist, [lanebase + b], zeros16i, mask=ones_m)
        return carry

    lax.fori_loop(0, NBUCKETS, _hzero_body, 0)

    def _hist_body(i, carry):
        fv = flat_v[pl.ds(i * 16, 16)]
        ch = lax.shift_right_logical(fv, SHIFT)
        valid = ch < NBUCKETS
        chc = jnp.minimum(ch, NBUCKETS - 1)
        cur = plsc.load_gather(hist, [lanebase + chc], mask=valid)
        plsc.store_scatter(hist, [lanebase + chc], cur + 1, mask=valid)
        return carry

    lax.fori_loop(0, PT // 16, _hist_body, 0)

    def _off_body(b, carry):
        bb = zeros16i + b
        cv = plsc.load_gather(hist, [lanebase + bb], mask=ones_m)
        inc = plsc.cumsum(cv)
        total = inc[15]
        plsc.store_scatter(tails, [lanebase + bb], carry + inc - cv,
                           mask=ones_m)
        lane0 = iota < 1
        plsc.store_scatter(starts, [bb], zeros16i + carry, mask=lane0)
        plsc.store_scatter(bcount, [bb], zeros16i + total, mask=lane0)
        return carry + ((total // WAVE) + 1) * WAVE

    lax.fori_loop(0, NBUCKETS, _off_body, 0)

    def _place_body(i, carry):
        fv = flat_v[pl.ds(i * 16, 16)]
        ch = lax.shift_right_logical(fv, SHIFT)
        valid = ch < NBUCKETS
        chc = jnp.minimum(ch, NBUCKETS - 1)
        dest = plsc.load_gather(tails, [lanebase + chc], mask=valid)
        plsc.store_scatter(sp_list, [dest], iota + (base + i * 16),
                           mask=valid)
        plsc.store_scatter(tails, [lanebase + chc], dest + 1, mask=valid)
        return carry

    lax.fori_loop(0, PT // 16, _place_body, 0)

    dummyv = zeros16i + (base + DUMMY_FLAG)

    def _fill_body(b, carry):
        bb = zeros16i + b
        st = plsc.load_gather(starts, [bb])[0]
        cnt = plsc.load_gather(bcount, [bb])[0]
        t = st + cnt
        end = st + ((cnt + WAVE - 1) // WAVE) * WAVE
        for k in range(WAVE // 16):
            offs = t + k * 16 + iota
            mfill = offs < end
            plsc.store_scatter(sp_list, [offs], dummyv, mask=mfill)
        return carry

    lax.fori_loop(0, NBUCKETS, _fill_body, 0)

    zrows = zbuf.shape[0]

    def _stage(buf_cell, buf_pid, off, nvreg):
        for k in range(nvreg):
            pidv = sp_list[pl.ds(off + k * 16, 16)]
            isd = pidv >= DUMMY_FLAG
            rp = pidv & (DUMMY_FLAG - 1)
            buf_pid[pl.ds(k * 16, 16)] = rp
            fl = plsc.load_gather(flat_v, [rp - base])
            buf_cell[pl.ds(k * 16, 16)] = jnp.where(
                isd, DUMMY_CELL, fl & (CHUNK - 1))

    def _pass_body(p, carry):
        chunk_id = c * PASSES + p
        lo = chunk_id * CHUNK
        bb = zeros16i + chunk_id
        st = pl.multiple_of(plsc.load_gather(starts, [bb])[0], WAVE)
        cnt = plsc.load_gather(bcount, [bb])[0]
        n_waves = (cnt + WAVE - 1) // WAVE

        def _clear_body(q, carry2):
            pltpu.sync_copy(
                zbuf, chunk.at[pl.ds(s * ROWS_PER_SUB + q * zrows, zrows)])
            return carry2

        lax.fori_loop(0, ROWS_PER_SUB // zrows, _clear_body, 0)
        plsc.subcore_barrier()

        n_big = cnt // BIGWAVE
        bb_cell = (cell_bb0, cell_bb1)
        bb_pid = (pid_bb0, pid_bb1)
        bb_rows = (rows_b0, rows_b1)
        bb_gsem = (gsem0, gsem1)
        bb_asem = (asem0, asem1)

        def _bigpair(jj, carry2):
            for par in range(2):
                j = jj * 2 + par

                @pl.when(j < n_big)
                def _(par=par, j=j):
                    @pl.when(j >= 2)
                    def _():
                        pltpu.make_async_copy(
                            bb_rows[par], chunk.at[bb_cell[par]],
                            bb_asem[par]).wait()

                    _stage(bb_cell[par], bb_pid[par], st + j * BIGWAVE,
                           BIGWAVE // 16)
                    pltpu.async_copy(h_hbm.at[bb_pid[par]], bb_rows[par],
                                     bb_gsem[par])
            for par in range(2):
                j = jj * 2 + par

                @pl.when(j < n_big)
                def _(par=par, j=j):
                    pltpu.make_async_copy(h_hbm.at[bb_pid[par]],
                                          bb_rows[par], bb_gsem[par]).wait()
                    pltpu.async_copy(bb_rows[par], chunk.at[bb_cell[par]],
                                     bb_asem[par], add=True)
            return carry2

        lax.fori_loop(0, (n_big + 1) // 2, _bigpair, 0)
        for par in range(2):

            @pl.when(n_big > par)
            def _(par=par):
                pltpu.make_async_copy(bb_rows[par], chunk.at[bb_cell[par]],
                                      bb_asem[par]).wait()

        def _wave_body(j, carry2):
            _stage(cell_buf, pid_buf, st + j * WAVE, WAVE // 16)
            pltpu.sync_copy(h_hbm.at[pid_buf], rows_v)
            pltpu.sync_copy(rows_v, chunk.at[cell_buf], add=True)
            return carry2

        lax.fori_loop(n_big * (BIGWAVE // WAVE), n_waves, _wave_body, 0)
        plsc.subcore_barrier()

        pltpu.sync_copy(
            chunk.at[pl.ds(s * ROWS_PER_SUB, ROWS_PER_SUB)],
            out_hbm.at[pl.ds(lo + s * ROWS_PER_SUB, ROWS_PER_SUB)])
        plsc.subcore_barrier()
        return carry

    lax.fori_loop(0, PASSES, _pass_body, 0)


def kernel(x, indices, W, b, gamma, beta):
    x = x.astype(jnp.float32)
    indices = indices.astype(jnp.int32)
    xi = jnp.pad(indices[:, 0], (0, P_PAD - N_POINTS),
                 constant_values=1 << 20)
    yi = jnp.pad(indices[:, 1], (0, P_PAD - N_POINTS))
    xc = jnp.pad(x, ((0, P_PAD - N_POINTS), (0, 0))).reshape(XROWS, PACKW)
    wt = W.astype(jnp.float32).T
    b2d = b.astype(jnp.float32).reshape(1, OUT_F)
    g2d = gamma.astype(jnp.float32).reshape(1, OUT_F)
    beta2d = beta.astype(jnp.float32).reshape(1, OUT_F)

    sx, sxx = pl.pallas_call(
        _stats_kernel,
        out_shape=[jax.ShapeDtypeStruct((1, IN_F), jnp.float32),
                   jax.ShapeDtypeStruct((IN_F, IN_F), jnp.float32)],
    )(xc)

    w2, b2 = pl.pallas_call(
        _fold_kernel,
        out_shape=[jax.ShapeDtypeStruct((IN_F, OUT_F), jnp.float32),
                   jax.ShapeDtypeStruct((1, OUT_F), jnp.float32)],
    )(sx, sxx, wt, b2d, g2d, beta2d)

    w384 = jnp.kron(jnp.eye(64, dtype=jnp.float32), w2)
    b4096 = jnp.tile(b2, (1, 64))

    h4 = pl.pallas_call(
        _forward_kernel,
        grid=(N_FBLOCKS,),
        in_specs=[pl.BlockSpec((FBK, PACKW), lambda i: (i, 0)),
                  pl.BlockSpec((PACKW, 64 * OUT_F), lambda i: (0, 0)),
                  pl.BlockSpec((1, 64 * OUT_F), lambda i: (0, 0))],
        out_specs=pl.BlockSpec((FBK, 64 * OUT_F), lambda i: (i, 0)),
        out_shape=jax.ShapeDtypeStruct((XROWS, 64 * OUT_F), jnp.float32),
    )(xc, w384, b4096)
    h = h4.reshape(P_PAD, OUT_F)

    mesh = plsc.VectorSubcoreMesh(core_axis_name="c", subcore_axis_name="s")
    grid_flat = pl.kernel(
        _scatter_kernel,
        mesh=mesh,
        out_type=jax.ShapeDtypeStruct((N_PX * N_PY, OUT_F), jnp.float32),
        compiler_params=pltpu.CompilerParams(use_tc_tiling_on_sc=False,
                                             needs_layout_passes=False),
        scratch_types=[
            pltpu.VMEM((LIST_CAP,), jnp.int32),
            pltpu.VMEM((PT,), jnp.int32),
            pltpu.VMEM((16 * NBUCKETS,), jnp.int32),
            pltpu.VMEM((16 * NBUCKETS,), jnp.int32),
            pltpu.VMEM((NBUCKETS,), jnp.int32),
            pltpu.VMEM((NBUCKETS,), jnp.int32),
            pltpu.VMEM((WAVE,), jnp.int32),
            pltpu.VMEM((WAVE,), jnp.int32),
            pltpu.VMEM((WAVE, OUT_F), jnp.float32),
            pltpu.VMEM((BIGWAVE,), jnp.int32),
            pltpu.VMEM((BIGWAVE,), jnp.int32),
            pltpu.VMEM((BIGWAVE, OUT_F), jnp.float32),
            pltpu.VMEM((BIGWAVE,), jnp.int32),
            pltpu.VMEM((BIGWAVE,), jnp.int32),
            pltpu.VMEM((BIGWAVE, OUT_F), jnp.float32),
            pltpu.SemaphoreType.DMA,
            pltpu.SemaphoreType.DMA,
            pltpu.SemaphoreType.DMA,
            pltpu.SemaphoreType.DMA,
            pltpu.VMEM((128, OUT_F), jnp.float32),
            pltpu.VMEM_SHARED((CHUNK + 8, OUT_F), jnp.float32),
        ],
    )(xi, yi, h)

    return grid_flat.reshape(N_PX, N_PY, OUT_F)


# speedup vs baseline: 2.5601x; 1.1826x over previous
import jax
import jax.numpy as jnp
from jax import lax
from jax.experimental import pallas as pl
from jax.experimental.pallas import tpu as pltpu
from jax.experimental.pallas import tpu_sc as plsc

N_POINTS = 200000
N_PX = 512
N_PY = 512
IN_F = 6
OUT_F = 64
EPS = 1e-5

P_PAD = 200704
FBK = 4096


NCORE = 2
NSUB = 16
PT = P_PAD // NSUB
CHUNK = 8192
SHIFT = 13
NBUCKETS = (N_PX * N_PY) // CHUNK
PASSES = NBUCKETS // NCORE
WAVE = 128
BIGWAVE = 256
LIST_CAP = PT + NBUCKETS * WAVE
ROWS_PER_SUB = CHUNK // NSUB
DUMMY_CELL = CHUNK
DUMMY_FLAG = 1 << 19


def _stats_kernel(xt_ref, sx_ref, sxx_ref):
    xt = xt_ref[...]
    sx_ref[...] = jnp.sum(xt, axis=1, keepdims=True)
    for f in range(IN_F):
        row = xt[f:f + 1, :]
        prods = row * xt
        sxx_ref[f:f + 1, :] = jnp.sum(prods, axis=1, keepdims=True).T


def _fold_kernel(sx_ref, sxx_ref, wt_ref, b_ref, g_ref, beta_ref,
                 w2_ref, b2_ref):
    mu = sx_ref[...].T / N_POINTS
    m2 = sxx_ref[...] / N_POINTS
    wt = wt_ref[...]
    b = b_ref[...]
    mean = lax.dot_general(mu, wt, (((1,), (0,)), ((), ())),
                           preferred_element_type=jnp.float32) + b
    m2w = lax.dot_general(m2, wt, (((1,), (0,)), ((), ())),
                          preferred_element_type=jnp.float32)
    quad = jnp.sum(wt * m2w, axis=0, keepdims=True)
    var = quad + 2.0 * b * mean - b * b - mean * mean
    sc = g_ref[...] * lax.rsqrt(var + EPS)
    w2_ref[...] = wt * sc
    b2_ref[...] = (b - mean) * sc + beta_ref[...]


def _forward_kernel(xt_ref, w2_ref, b2_ref, h_ref):
    h = lax.dot_general(xt_ref[...], w2_ref[...], (((0,), (0,)), ((), ())),
                        preferred_element_type=jnp.float32)
    h_ref[...] = jnp.maximum(h + b2_ref[...], 0.0)


def _scatter_kernel(xi_hbm, yi_hbm, h_hbm, out_hbm,
                    sp_list, flat_v, hist, tails, starts, bcount,
                    cell_buf, pid_buf, rows_v, cell_bb0, pid_bb0, rows_b0,
                    cell_bb1, pid_bb1, rows_b1, gsem0, gsem1, asem0, asem1,
                    zbuf, chunk):
    c = lax.axis_index("c")
    s = lax.axis_index("s")
    base = s * PT
    iota = lax.iota(jnp.int32, 16)
    zeros16i = jnp.zeros((16,), jnp.int32)
    ones_m = iota < 16
    lanebase = iota * NBUCKETS

    pltpu.sync_copy(xi_hbm.at[pl.ds(base, PT)], flat_v)
    pltpu.sync_copy(yi_hbm.at[pl.ds(base, PT)], sp_list.at[pl.ds(0, PT)])

    def _flat_body(i, carry):
        xv = flat_v[pl.ds(i * 16, 16)]
        yv = sp_list[pl.ds(i * 16, 16)]
        flat_v[pl.ds(i * 16, 16)] = xv * N_PY + yv
        return carry

    lax.fori_loop(0, PT // 16, _flat_body, 0)

    zeros16f = jnp.zeros((16,), jnp.float32)

    def _zero_body(i, carry):
        for j in range(OUT_F // 16):
            zbuf[i, pl.ds(j * 16, 16)] = zeros16f
        return carry

    lax.fori_loop(0, zbuf.shape[0], _zero_body, 0)

    def _hzero_body(b, carry):
        plsc.store_scatter(hist, [lanebase + b], zeros16i, mask=ones_m)
        return carry

    lax.fori_loop(0, NBUCKETS, _hzero_body, 0)

    def _hist_body(i, carry):
        fv = flat_v[pl.ds(i * 16, 16)]
        ch = lax.shift_right_logical(fv, SHIFT)
        valid = ch < NBUCKETS
        chc = jnp.minimum(ch, NBUCKETS - 1)
        cur = plsc.load_gather(hist, [lanebase + chc], mask=valid)
        plsc.store_scatter(hist, [lanebase + chc], cur + 1, mask=valid)
        return carry

    lax.fori_loop(0, PT // 16, _hist_body, 0)

    def _off_body(b, carry):
        bb = zeros16i + b
        cv = plsc.load_gather(hist, [lanebase + bb], mask=ones_m)
        inc = plsc.cumsum(cv)
        total = inc[15]
        plsc.store_scatter(tails, [lanebase + bb], carry + inc - cv,
                           mask=ones_m)
        lane0 = iota < 1
        plsc.store_scatter(starts, [bb], zeros16i + carry, mask=lane0)
        plsc.store_scatter(bcount, [bb], zeros16i + total, mask=lane0)
        return carry + ((total // WAVE) + 1) * WAVE

    lax.fori_loop(0, NBUCKETS, _off_body, 0)

    def _place_body(i, carry):
        fv = flat_v[pl.ds(i * 16, 16)]
        ch = lax.shift_right_logical(fv, SHIFT)
        valid = ch < NBUCKETS
        chc = jnp.minimum(ch, NBUCKETS - 1)
        dest = plsc.load_gather(tails, [lanebase + chc], mask=valid)
        plsc.store_scatter(sp_list, [dest], iota + (base + i * 16),
                           mask=valid)
        plsc.store_scatter(tails, [lanebase + chc], dest + 1, mask=valid)
        return carry

    lax.fori_loop(0, PT // 16, _place_body, 0)

    dummyv = zeros16i + (base + DUMMY_FLAG)

    def _fill_body(b, carry):
        bb = zeros16i + b
        st = plsc.load_gather(starts, [bb])[0]
        cnt = plsc.load_gather(bcount, [bb])[0]
        t = st + cnt
        end = st + ((cnt + WAVE - 1) // WAVE) * WAVE
        for k in range(WAVE // 16):
            offs = t + k * 16 + iota
            mfill = offs < end
            plsc.store_scatter(sp_list, [offs], dummyv, mask=mfill)
        return carry

    lax.fori_loop(0, NBUCKETS, _fill_body, 0)

    zrows = zbuf.shape[0]

    def _stage(buf_cell, buf_pid, off, nvreg):
        for k in range(nvreg):
            pidv = sp_list[pl.ds(off + k * 16, 16)]
            isd = pidv >= DUMMY_FLAG
            rp = pidv & (DUMMY_FLAG - 1)
            buf_pid[pl.ds(k * 16, 16)] = rp
            fl = plsc.load_gather(flat_v, [rp - base])
            buf_cell[pl.ds(k * 16, 16)] = jnp.where(
                isd, DUMMY_CELL, fl & (CHUNK - 1))

    def _pass_body(p, carry):
        chunk_id = c * PASSES + p
        lo = chunk_id * CHUNK
        bb = zeros16i + chunk_id
        st = pl.multiple_of(plsc.load_gather(starts, [bb])[0], WAVE)
        cnt = plsc.load_gather(bcount, [bb])[0]
        n_waves = (cnt + WAVE - 1) // WAVE

        def _clear_body(q, carry2):
            pltpu.sync_copy(
                zbuf, chunk.at[pl.ds(s * ROWS_PER_SUB + q * zrows, zrows)])
            return carry2

        lax.fori_loop(0, ROWS_PER_SUB // zrows, _clear_body, 0)
        plsc.subcore_barrier()

        n_big = cnt // BIGWAVE
        bb_cell = (cell_bb0, cell_bb1)
        bb_pid = (pid_bb0, pid_bb1)
        bb_rows = (rows_b0, rows_b1)
        bb_gsem = (gsem0, gsem1)
        bb_asem = (asem0, asem1)

        def _bigpair(jj, carry2):
            for par in range(2):
                j = jj * 2 + par

                @pl.when(j < n_big)
                def _(par=par, j=j):
                    @pl.when(j >= 2)
                    def _():
                        pltpu.make_async_copy(
                            bb_rows[par], chunk.at[bb_cell[par]],
                            bb_asem[par]).wait()

                    _stage(bb_cell[par], bb_pid[par], st + j * BIGWAVE,
                           BIGWAVE // 16)
                    pltpu.async_copy(h_hbm.at[bb_pid[par]], bb_rows[par],
                                     bb_gsem[par])
            for par in range(2):
                j = jj * 2 + par

                @pl.when(j < n_big)
                def _(par=par, j=j):
                    pltpu.make_async_copy(h_hbm.at[bb_pid[par]],
                                          bb_rows[par], bb_gsem[par]).wait()
                    pltpu.async_copy(bb_rows[par], chunk.at[bb_cell[par]],
                                     bb_asem[par], add=True)
            return carry2

        lax.fori_loop(0, (n_big + 1) // 2, _bigpair, 0)
        for par in range(2):

            @pl.when(n_big > par)
            def _(par=par):
                pltpu.make_async_copy(bb_rows[par], chunk.at[bb_cell[par]],
                                      bb_asem[par]).wait()

        def _wave_body(j, carry2):
            _stage(cell_buf, pid_buf, st + j * WAVE, WAVE // 16)
            pltpu.sync_copy(h_hbm.at[pid_buf], rows_v)
            pltpu.sync_copy(rows_v, chunk.at[cell_buf], add=True)
            return carry2

        lax.fori_loop(n_big * (BIGWAVE // WAVE), n_waves, _wave_body, 0)
        plsc.subcore_barrier()

        pltpu.sync_copy(
            chunk.at[pl.ds(s * ROWS_PER_SUB, ROWS_PER_SUB)],
            out_hbm.at[pl.ds(lo + s * ROWS_PER_SUB, ROWS_PER_SUB)])
        plsc.subcore_barrier()
        return carry

    lax.fori_loop(0, PASSES, _pass_body, 0)


def kernel(x, indices, W, b, gamma, beta):
    x = x.astype(jnp.float32)
    indices = indices.astype(jnp.int32)
    xi = jnp.pad(indices[:, 0], (0, P_PAD - N_POINTS),
                 constant_values=1 << 20)
    yi = jnp.pad(indices[:, 1], (0, P_PAD - N_POINTS))
    xt = jnp.pad(x, ((0, P_PAD - N_POINTS), (0, 0))).T
    wt = W.astype(jnp.float32).T
    b2d = b.astype(jnp.float32).reshape(1, OUT_F)
    g2d = gamma.astype(jnp.float32).reshape(1, OUT_F)
    beta2d = beta.astype(jnp.float32).reshape(1, OUT_F)

    sx, sxx = pl.pallas_call(
        _stats_kernel,
        out_shape=[jax.ShapeDtypeStruct((IN_F, 1), jnp.float32),
                   jax.ShapeDtypeStruct((IN_F, IN_F), jnp.float32)],
    )(xt)

    w2, b2 = pl.pallas_call(
        _fold_kernel,
        out_shape=[jax.ShapeDtypeStruct((IN_F, OUT_F), jnp.float32),
                   jax.ShapeDtypeStruct((1, OUT_F), jnp.float32)],
    )(sx, sxx, wt, b2d, g2d, beta2d)

    h = pl.pallas_call(
        _forward_kernel,
        grid=(P_PAD // FBK,),
        in_specs=[pl.BlockSpec((IN_F, FBK), lambda i: (0, i)),
                  pl.BlockSpec((IN_F, OUT_F), lambda i: (0, 0)),
                  pl.BlockSpec((1, OUT_F), lambda i: (0, 0))],
        out_specs=pl.BlockSpec((FBK, OUT_F), lambda i: (i, 0)),
        out_shape=jax.ShapeDtypeStruct((P_PAD, OUT_F), jnp.float32),
    )(xt, w2, b2)

    mesh = plsc.VectorSubcoreMesh(core_axis_name="c", subcore_axis_name="s")
    grid_flat = pl.kernel(
        _scatter_kernel,
        mesh=mesh,
        out_type=jax.ShapeDtypeStruct((N_PX * N_PY, OUT_F), jnp.float32),
        compiler_params=pltpu.CompilerParams(use_tc_tiling_on_sc=False,
                                             needs_layout_passes=False),
        scratch_types=[
            pltpu.VMEM((LIST_CAP,), jnp.int32),
            pltpu.VMEM((PT,), jnp.int32),
            pltpu.VMEM((16 * NBUCKETS,), jnp.int32),
            pltpu.VMEM((16 * NBUCKETS,), jnp.int32),
            pltpu.VMEM((NBUCKETS,), jnp.int32),
            pltpu.VMEM((NBUCKETS,), jnp.int32),
            pltpu.VMEM((WAVE,), jnp.int32),
            pltpu.VMEM((WAVE,), jnp.int32),
            pltpu.VMEM((WAVE, OUT_F), jnp.float32),
            pltpu.VMEM((BIGWAVE,), jnp.int32),
            pltpu.VMEM((BIGWAVE,), jnp.int32),
            pltpu.VMEM((BIGWAVE, OUT_F), jnp.float32),
            pltpu.VMEM((BIGWAVE,), jnp.int32),
            pltpu.VMEM((BIGWAVE,), jnp.int32),
            pltpu.VMEM((BIGWAVE, OUT_F), jnp.float32),
            pltpu.SemaphoreType.DMA,
            pltpu.SemaphoreType.DMA,
            pltpu.SemaphoreType.DMA,
            pltpu.SemaphoreType.DMA,
            pltpu.VMEM((128, OUT_F), jnp.float32),
            pltpu.VMEM_SHARED((CHUNK + 8, OUT_F), jnp.float32),
        ],
    )(xi, yi, h)

    return grid_flat.reshape(N_PX, N_PY, OUT_F)


# speedup vs baseline: 3.0406x; 1.1877x over previous
import jax
import jax.numpy as jnp
from jax import lax
from jax.experimental import pallas as pl
from jax.experimental.pallas import tpu as pltpu
from jax.experimental.pallas import tpu_sc as plsc

N_POINTS = 200000
N_PX = 512
N_PY = 512
IN_F = 6
OUT_F = 64
EPS = 1e-5

P_PAD = 200704
FBK = 4096


NCORE = 2
NSUB = 16
PT = P_PAD // NSUB
CHUNK = 8192
SHIFT = 13
NBUCKETS = (N_PX * N_PY) // CHUNK
PASSES = NBUCKETS // NCORE
WAVE = 128
BIGWAVE = 256
LIST_CAP = PT + NBUCKETS * WAVE
ROWS_PER_SUB = CHUNK // NSUB
DUMMY_CELL = CHUNK
DUMMY_FLAG = 1 << 19


def _stats_kernel(xt_ref, sx_ref, sxx_ref):
    xt = xt_ref[...]
    sx_ref[...] = jnp.sum(xt, axis=1, keepdims=True)
    for f in range(IN_F):
        row = xt[f:f + 1, :]
        prods = row * xt
        sxx_ref[f:f + 1, :] = jnp.sum(prods, axis=1, keepdims=True).T


def _fold_kernel(sx_ref, sxx_ref, wt_ref, b_ref, g_ref, beta_ref,
                 w2_ref, b2_ref):
    mu = sx_ref[...].T / N_POINTS
    m2 = sxx_ref[...] / N_POINTS
    wt = wt_ref[...]
    b = b_ref[...]
    mean = lax.dot_general(mu, wt, (((1,), (0,)), ((), ())),
                           preferred_element_type=jnp.float32) + b
    m2w = lax.dot_general(m2, wt, (((1,), (0,)), ((), ())),
                          preferred_element_type=jnp.float32)
    quad = jnp.sum(wt * m2w, axis=0, keepdims=True)
    var = quad + 2.0 * b * mean - b * b - mean * mean
    sc = g_ref[...] * lax.rsqrt(var + EPS)
    w2_ref[...] = wt * sc
    b2_ref[...] = (b - mean) * sc + beta_ref[...]


def _forward_kernel(xt_ref, w2_ref, b2_ref, h_ref):
    h = lax.dot_general(xt_ref[...], w2_ref[...], (((0,), (0,)), ((), ())),
                        preferred_element_type=jnp.float32)
    h_ref[...] = jnp.maximum(h + b2_ref[...], 0.0)


def _scatter_kernel(xi_hbm, yi_hbm, h_hbm, out_hbm,
                    sp_list, flat_v, hist, tails, starts, bcount,
                    cell_buf, pid_buf, rows_v, cell_bb0, pid_bb0, rows_b0,
                    cell_bb1, pid_bb1, rows_b1, gsem0, gsem1, asem0, asem1,
                    zbuf, chunk):
    c = lax.axis_index("c")
    s = lax.axis_index("s")
    base = s * PT
    iota = lax.iota(jnp.int32, 16)
    zeros16i = jnp.zeros((16,), jnp.int32)
    ones_m = iota < 16
    lanebase = iota * NBUCKETS

    pltpu.sync_copy(xi_hbm.at[pl.ds(base, PT)], flat_v)
    pltpu.sync_copy(yi_hbm.at[pl.ds(base, PT)], sp_list.at[pl.ds(0, PT)])

    def _flat_body(i, carry):
        xv = flat_v[pl.ds(i * 16, 16)]
        yv = sp_list[pl.ds(i * 16, 16)]
        flat_v[pl.ds(i * 16, 16)] = xv * N_PY + yv
        return carry

    lax.fori_loop(0, PT // 16, _flat_body, 0)

    zeros16f = jnp.zeros((16,), jnp.float32)

    def _zero_body(i, carry):
        for j in range(OUT_F // 16):
            zbuf[i, pl.ds(j * 16, 16)] = zeros16f
        return carry

    lax.fori_loop(0, zbuf.shape[0], _zero_body, 0)

    def _hzero_body(b, carry):
        plsc.store_scatter(hist, [lanebase + b], zeros16i, mask=ones_m)
        return carry

    lax.fori_loop(0, NBUCKETS, _hzero_body, 0)

    def _hist_body(i, carry):
        fv = flat_v[pl.ds(i * 16, 16)]
        ch = lax.shift_right_logical(fv, SHIFT)
        valid = ch < NBUCKETS
        chc = jnp.minimum(ch, NBUCKETS - 1)
        cur = plsc.load_gather(hist, [lanebase + chc], mask=valid)
        plsc.store_scatter(hist, [lanebase + chc], cur + 1, mask=valid)
        return carry

    lax.fori_loop(0, PT // 16, _hist_body, 0)

    def _off_body(b, carry):
        bb = zeros16i + b
        cv = plsc.load_gather(hist, [lanebase + bb], mask=ones_m)
        inc = plsc.cumsum(cv)
        total = inc[15]
        plsc.store_scatter(tails, [lanebase + bb], carry + inc - cv,
                           mask=ones_m)
        lane0 = iota < 1
        plsc.store_scatter(starts, [bb], zeros16i + carry, mask=lane0)
        plsc.store_scatter(bcount, [bb], zeros16i + total, mask=lane0)
        return carry + ((total // WAVE) + 1) * WAVE

    lax.fori_loop(0, NBUCKETS, _off_body, 0)

    def _place_body(i, carry):
        fv = flat_v[pl.ds(i * 16, 16)]
        ch = lax.shift_right_logical(fv, SHIFT)
        valid = ch < NBUCKETS
        chc = jnp.minimum(ch, NBUCKETS - 1)
        dest = plsc.load_gather(tails, [lanebase + chc], mask=valid)
        plsc.store_scatter(sp_list, [dest], iota + (base + i * 16),
                           mask=valid)
        plsc.store_scatter(tails, [lanebase + chc], dest + 1, mask=valid)
        return carry

    lax.fori_loop(0, PT // 16, _place_body, 0)

    dummyv = zeros16i + (base + DUMMY_FLAG)

    def _fill_body(b, carry):
        bb = zeros16i + b
        st = plsc.load_gather(starts, [bb])[0]
        cnt = plsc.load_gather(bcount, [bb])[0]
        t = st + cnt
        end = st + ((cnt + WAVE - 1) // WAVE) * WAVE
        for k in range(WAVE // 16):
            offs = t + k * 16 + iota
            mfill = offs < end
            plsc.store_scatter(sp_list, [offs], dummyv, mask=mfill)
        return carry

    lax.fori_loop(0, NBUCKETS, _fill_body, 0)

    zrows = zbuf.shape[0]

    def _stage(buf_cell, buf_pid, off, nvreg):
        for k in range(nvreg):
            pidv = sp_list[pl.ds(off + k * 16, 16)]
            isd = pidv >= DUMMY_FLAG
            rp = pidv & (DUMMY_FLAG - 1)
            buf_pid[pl.ds(k * 16, 16)] = rp
            fl = plsc.load_gather(flat_v, [rp - base])
            buf_cell[pl.ds(k * 16, 16)] = jnp.where(
                isd, DUMMY_CELL, fl & (CHUNK - 1))

    def _pass_body(p, carry):
        chunk_id = c * PASSES + p
        lo = chunk_id * CHUNK
        bb = zeros16i + chunk_id
        st = pl.multiple_of(plsc.load_gather(starts, [bb])[0], WAVE)
        cnt = plsc.load_gather(bcount, [bb])[0]
        n_waves = (cnt + WAVE - 1) // WAVE

        def _clear_body(q, carry2):
            pltpu.sync_copy(
                zbuf, chunk.at[pl.ds(s * ROWS_PER_SUB + q * zrows, zrows)])
            return carry2

        lax.fori_loop(0, ROWS_PER_SUB // zrows, _clear_body, 0)
        plsc.subcore_barrier()

        n_big = cnt // BIGWAVE
        bb_cell = (cell_bb0, cell_bb1)
        bb_pid = (pid_bb0, pid_bb1)
        bb_rows = (rows_b0, rows_b1)
        bb_gsem = (gsem0, gsem1)
        bb_asem = (asem0, asem1)

        def _bigpair(jj, carry2):
            for par in range(2):
                j = jj * 2 + par

                @pl.when(j < n_big)
                def _(par=par, j=j):
                    @pl.when(j >= 2)
                    def _():
                        pltpu.make_async_copy(
                            bb_rows[par], chunk.at[bb_cell[par]],
                            bb_asem[par]).wait()

                    _stage(bb_cell[par], bb_pid[par], st + j * BIGWAVE,
                           BIGWAVE // 16)
                    pltpu.async_copy(h_hbm.at[bb_pid[par]], bb_rows[par],
                                     bb_gsem[par])
            for par in range(2):
                j = jj * 2 + par

                @pl.when(j < n_big)
                def _(par=par, j=j):
                    pltpu.make_async_copy(h_hbm.at[bb_pid[par]],
                                          bb_rows[par], bb_gsem[par]).wait()
                    pltpu.async_copy(bb_rows[par], chunk.at[bb_cell[par]],
                                     bb_asem[par], add=True)
            return carry2

        lax.fori_loop(0, (n_big + 1) // 2, _bigpair, 0)
        for par in range(2):

            @pl.when(n_big > par)
            def _(par=par):
                pltpu.make_async_copy(bb_rows[par], chunk.at[bb_cell[par]],
                                      bb_asem[par]).wait()

        def _wave_body(j, carry2):
            _stage(cell_buf, pid_buf, st + j * WAVE, WAVE // 16)
            pltpu.sync_copy(h_hbm.at[pid_buf], rows_v)
            pltpu.sync_copy(rows_v, chunk.at[cell_buf], add=True)
            return carry2

        lax.fori_loop(n_big * (BIGWAVE // WAVE), n_waves, _wave_body, 0)
        plsc.subcore_barrier()

        pltpu.sync_copy(
            chunk.at[pl.ds(s * ROWS_PER_SUB, ROWS_PER_SUB)],
            out_hbm.at[pl.ds(lo + s * ROWS_PER_SUB, ROWS_PER_SUB),
                       pl.ds(0, OUT_F)])
        plsc.subcore_barrier()
        return carry

    lax.fori_loop(0, PASSES, _pass_body, 0)


def kernel(x, indices, W, b, gamma, beta):
    x = x.astype(jnp.float32)
    indices = indices.astype(jnp.int32)
    xi = jnp.pad(indices[:, 0], (0, P_PAD - N_POINTS),
                 constant_values=1 << 20)
    yi = jnp.pad(indices[:, 1], (0, P_PAD - N_POINTS))
    xt = jnp.pad(x, ((0, P_PAD - N_POINTS), (0, 0))).T
    wt = W.astype(jnp.float32).T
    b2d = b.astype(jnp.float32).reshape(1, OUT_F)
    g2d = gamma.astype(jnp.float32).reshape(1, OUT_F)
    beta2d = beta.astype(jnp.float32).reshape(1, OUT_F)

    sx, sxx = pl.pallas_call(
        _stats_kernel,
        out_shape=[jax.ShapeDtypeStruct((IN_F, 1), jnp.float32),
                   jax.ShapeDtypeStruct((IN_F, IN_F), jnp.float32)],
    )(xt)

    w2, b2 = pl.pallas_call(
        _fold_kernel,
        out_shape=[jax.ShapeDtypeStruct((IN_F, OUT_F), jnp.float32),
                   jax.ShapeDtypeStruct((1, OUT_F), jnp.float32)],
    )(sx, sxx, wt, b2d, g2d, beta2d)

    h = pl.pallas_call(
        _forward_kernel,
        grid=(P_PAD // FBK,),
        in_specs=[pl.BlockSpec((IN_F, FBK), lambda i: (0, i)),
                  pl.BlockSpec((IN_F, OUT_F), lambda i: (0, 0)),
                  pl.BlockSpec((1, OUT_F), lambda i: (0, 0))],
        out_specs=pl.BlockSpec((FBK, OUT_F), lambda i: (i, 0)),
        out_shape=jax.ShapeDtypeStruct((P_PAD, OUT_F), jnp.float32),
    )(xt, w2, b2)

    mesh = plsc.VectorSubcoreMesh(core_axis_name="c", subcore_axis_name="s")
    grid_flat = pl.kernel(
        _scatter_kernel,
        mesh=mesh,
        out_type=jax.ShapeDtypeStruct((N_PX * N_PY, 2 * OUT_F),
                                      jnp.float32),
        compiler_params=pltpu.CompilerParams(use_tc_tiling_on_sc=False,
                                             needs_layout_passes=False),
        scratch_types=[
            pltpu.VMEM((LIST_CAP,), jnp.int32),
            pltpu.VMEM((PT,), jnp.int32),
            pltpu.VMEM((16 * NBUCKETS,), jnp.int32),
            pltpu.VMEM((16 * NBUCKETS,), jnp.int32),
            pltpu.VMEM((NBUCKETS,), jnp.int32),
            pltpu.VMEM((NBUCKETS,), jnp.int32),
            pltpu.VMEM((WAVE,), jnp.int32),
            pltpu.VMEM((WAVE,), jnp.int32),
            pltpu.VMEM((WAVE, OUT_F), jnp.float32),
            pltpu.VMEM((BIGWAVE,), jnp.int32),
            pltpu.VMEM((BIGWAVE,), jnp.int32),
            pltpu.VMEM((BIGWAVE, OUT_F), jnp.float32),
            pltpu.VMEM((BIGWAVE,), jnp.int32),
            pltpu.VMEM((BIGWAVE,), jnp.int32),
            pltpu.VMEM((BIGWAVE, OUT_F), jnp.float32),
            pltpu.SemaphoreType.DMA,
            pltpu.SemaphoreType.DMA,
            pltpu.SemaphoreType.DMA,
            pltpu.SemaphoreType.DMA,
            pltpu.VMEM((128, OUT_F), jnp.float32),
            pltpu.VMEM_SHARED((CHUNK + 8, OUT_F), jnp.float32),
        ],
    )(xi, yi, h)

    return grid_flat[:, :OUT_F].reshape(N_PX, N_PY, OUT_F)


# speedup vs baseline: 3.0651x; 1.0080x over previous
import jax
import jax.numpy as jnp
from jax import lax
from jax.experimental import pallas as pl
from jax.experimental.pallas import tpu as pltpu
from jax.experimental.pallas import tpu_sc as plsc

N_POINTS = 200000
N_PX = 512
N_PY = 512
IN_F = 6
OUT_F = 64
EPS = 1e-5

P_PAD = 200704
FBK = 4096


NCORE = 2
NSUB = 16
PT = P_PAD // NSUB
CHUNK = 8192
SHIFT = 13
NBUCKETS = (N_PX * N_PY) // CHUNK
PASSES = NBUCKETS // NCORE
WAVE = 128
BIGWAVE = 256
LIST_CAP = PT + NBUCKETS * WAVE
ROWS_PER_SUB = CHUNK // NSUB
DUMMY_CELL = CHUNK
DUMMY_FLAG = 1 << 19


def _stats_fold_kernel(xt_ref, wt_ref, b_ref, g_ref, beta_ref,
                       w2_ref, b2_ref):
    xt = xt_ref[...]
    mu = jnp.sum(xt, axis=1, keepdims=True).T / N_POINTS
    m2 = jnp.concatenate(
        [jnp.sum(xt[f:f + 1, :] * xt, axis=1, keepdims=True).T
         for f in range(IN_F)], axis=0) / N_POINTS
    wt = wt_ref[...]
    b = b_ref[...]
    mean = lax.dot_general(mu, wt, (((1,), (0,)), ((), ())),
                           preferred_element_type=jnp.float32) + b
    m2w = lax.dot_general(m2, wt, (((1,), (0,)), ((), ())),
                          preferred_element_type=jnp.float32)
    quad = jnp.sum(wt * m2w, axis=0, keepdims=True)
    var = quad + 2.0 * b * mean - b * b - mean * mean
    sc = g_ref[...] * lax.rsqrt(var + EPS)
    w2_ref[...] = wt * sc
    b2_ref[...] = (b - mean) * sc + beta_ref[...]


def _forward_kernel(xt_ref, w2_ref, b2_ref, h_ref):
    h = lax.dot_general(xt_ref[...], w2_ref[...], (((0,), (0,)), ((), ())),
                        preferred_element_type=jnp.float32)
    h_ref[...] = jnp.maximum(h + b2_ref[...], 0.0)


def _scatter_kernel(xi_hbm, yi_hbm, h_hbm, out_hbm,
                    sp_list, flat_v, hist, tails, starts, bcount,
                    cell_buf, pid_buf, rows_v, cell_bb0, pid_bb0, rows_b0,
                    cell_bb1, pid_bb1, rows_b1, gsem0, gsem1, asem0, asem1,
                    zbuf, chunk):
    c = lax.axis_index("c")
    s = lax.axis_index("s")
    base = s * PT
    iota = lax.iota(jnp.int32, 16)
    zeros16i = jnp.zeros((16,), jnp.int32)
    ones_m = iota < 16
    lanebase = iota * NBUCKETS

    pltpu.sync_copy(xi_hbm.at[pl.ds(base, PT)], flat_v)
    pltpu.sync_copy(yi_hbm.at[pl.ds(base, PT)], sp_list.at[pl.ds(0, PT)])

    def _flat_body(i, carry):
        xv = flat_v[pl.ds(i * 16, 16)]
        yv = sp_list[pl.ds(i * 16, 16)]
        flat_v[pl.ds(i * 16, 16)] = xv * N_PY + yv
        return carry

    lax.fori_loop(0, PT // 16, _flat_body, 0)

    zeros16f = jnp.zeros((16,), jnp.float32)

    def _zero_body(i, carry):
        for j in range(OUT_F // 16):
            zbuf[i, pl.ds(j * 16, 16)] = zeros16f
        return carry

    lax.fori_loop(0, zbuf.shape[0], _zero_body, 0)

    def _hzero_body(b, carry):
        plsc.store_scatter(hist, [lanebase + b], zeros16i, mask=ones_m)
        return carry

    lax.fori_loop(0, NBUCKETS, _hzero_body, 0)

    def _hist_body(i, carry):
        fv = flat_v[pl.ds(i * 16, 16)]
        ch = lax.shift_right_logical(fv, SHIFT)
        valid = ch < NBUCKETS
        chc = jnp.minimum(ch, NBUCKETS - 1)
        cur = plsc.load_gather(hist, [lanebase + chc], mask=valid)
        plsc.store_scatter(hist, [lanebase + chc], cur + 1, mask=valid)
        return carry

    lax.fori_loop(0, PT // 16, _hist_body, 0)

    def _off_body(b, carry):
        bb = zeros16i + b
        cv = plsc.load_gather(hist, [lanebase + bb], mask=ones_m)
        inc = plsc.cumsum(cv)
        total = inc[15]
        plsc.store_scatter(tails, [lanebase + bb], carry + inc - cv,
                           mask=ones_m)
        lane0 = iota < 1
        plsc.store_scatter(starts, [bb], zeros16i + carry, mask=lane0)
        plsc.store_scatter(bcount, [bb], zeros16i + total, mask=lane0)
        return carry + ((total // WAVE) + 1) * WAVE

    lax.fori_loop(0, NBUCKETS, _off_body, 0)

    def _place_body(i, carry):
        fv = flat_v[pl.ds(i * 16, 16)]
        ch = lax.shift_right_logical(fv, SHIFT)
        valid = ch < NBUCKETS
        chc = jnp.minimum(ch, NBUCKETS - 1)
        dest = plsc.load_gather(tails, [lanebase + chc], mask=valid)
        plsc.store_scatter(sp_list, [dest], iota + (base + i * 16),
                           mask=valid)
        plsc.store_scatter(tails, [lanebase + chc], dest + 1, mask=valid)
        return carry

    lax.fori_loop(0, PT // 16, _place_body, 0)

    dummyv = zeros16i + (base + DUMMY_FLAG)

    def _fill_body(b, carry):
        bb = zeros16i + b
        st = plsc.load_gather(starts, [bb])[0]
        cnt = plsc.load_gather(bcount, [bb])[0]
        t = st + cnt
        end = st + ((cnt + WAVE - 1) // WAVE) * WAVE
        for k in range(WAVE // 16):
            offs = t + k * 16 + iota
            mfill = offs < end
            plsc.store_scatter(sp_list, [offs], dummyv, mask=mfill)
        return carry

    lax.fori_loop(0, NBUCKETS, _fill_body, 0)

    zrows = zbuf.shape[0]

    def _stage(buf_cell, buf_pid, off, nvreg):
        for k in range(nvreg):
            pidv = sp_list[pl.ds(off + k * 16, 16)]
            isd = pidv >= DUMMY_FLAG
            rp = pidv & (DUMMY_FLAG - 1)
            buf_pid[pl.ds(k * 16, 16)] = rp
            fl = plsc.load_gather(flat_v, [rp - base])
            buf_cell[pl.ds(k * 16, 16)] = jnp.where(
                isd, DUMMY_CELL, fl & (CHUNK - 1))

    def _pass_body(p, carry):
        chunk_id = c * PASSES + p
        lo = chunk_id * CHUNK
        bb = zeros16i + chunk_id
        st = pl.multiple_of(plsc.load_gather(starts, [bb])[0], WAVE)
        cnt = plsc.load_gather(bcount, [bb])[0]
        n_waves = (cnt + WAVE - 1) // WAVE

        def _clear_body(q, carry2):
            pltpu.sync_copy(
                zbuf, chunk.at[pl.ds(s * ROWS_PER_SUB + q * zrows, zrows)])
            return carry2

        lax.fori_loop(0, ROWS_PER_SUB // zrows, _clear_body, 0)
        plsc.subcore_barrier()

        n_big = cnt // BIGWAVE
        bb_cell = (cell_bb0, cell_bb1)
        bb_pid = (pid_bb0, pid_bb1)
        bb_rows = (rows_b0, rows_b1)
        bb_gsem = (gsem0, gsem1)
        bb_asem = (asem0, asem1)

        def _bigpair(jj, carry2):
            for par in range(2):
                j = jj * 2 + par

                @pl.when(j < n_big)
                def _(par=par, j=j):
                    @pl.when(j >= 2)
                    def _():
                        pltpu.make_async_copy(
                            bb_rows[par], chunk.at[bb_cell[par]],
                            bb_asem[par]).wait()

                    _stage(bb_cell[par], bb_pid[par], st + j * BIGWAVE,
                           BIGWAVE // 16)
                    pltpu.async_copy(h_hbm.at[bb_pid[par]], bb_rows[par],
                                     bb_gsem[par])
            for par in range(2):
                j = jj * 2 + par

                @pl.when(j < n_big)
                def _(par=par, j=j):
                    pltpu.make_async_copy(h_hbm.at[bb_pid[par]],
                                          bb_rows[par], bb_gsem[par]).wait()
                    pltpu.async_copy(bb_rows[par], chunk.at[bb_cell[par]],
                                     bb_asem[par], add=True)
            return carry2

        lax.fori_loop(0, (n_big + 1) // 2, _bigpair, 0)
        for par in range(2):

            @pl.when(n_big > par)
            def _(par=par):
                pltpu.make_async_copy(bb_rows[par], chunk.at[bb_cell[par]],
                                      bb_asem[par]).wait()

        def _wave_body(j, carry2):
            _stage(cell_buf, pid_buf, st + j * WAVE, WAVE // 16)
            pltpu.sync_copy(h_hbm.at[pid_buf], rows_v)
            pltpu.sync_copy(rows_v, chunk.at[cell_buf], add=True)
            return carry2

        lax.fori_loop(n_big * (BIGWAVE // WAVE), n_waves, _wave_body, 0)
        plsc.subcore_barrier()

        pltpu.sync_copy(
            chunk.at[pl.ds(s * ROWS_PER_SUB, ROWS_PER_SUB)],
            out_hbm.at[pl.ds(lo + s * ROWS_PER_SUB, ROWS_PER_SUB),
                       pl.ds(0, OUT_F)])
        plsc.subcore_barrier()
        return carry

    lax.fori_loop(0, PASSES, _pass_body, 0)


def kernel(x, indices, W, b, gamma, beta):
    x = x.astype(jnp.float32)
    indices = indices.astype(jnp.int32)
    xi = jnp.pad(indices[:, 0], (0, P_PAD - N_POINTS),
                 constant_values=1 << 20)
    yi = jnp.pad(indices[:, 1], (0, P_PAD - N_POINTS))
    xt = jnp.pad(x, ((0, P_PAD - N_POINTS), (0, 0))).T
    wt = W.astype(jnp.float32).T
    b2d = b.astype(jnp.float32).reshape(1, OUT_F)
    g2d = gamma.astype(jnp.float32).reshape(1, OUT_F)
    beta2d = beta.astype(jnp.float32).reshape(1, OUT_F)

    w2, b2 = pl.pallas_call(
        _stats_fold_kernel,
        out_shape=[jax.ShapeDtypeStruct((IN_F, OUT_F), jnp.float32),
                   jax.ShapeDtypeStruct((1, OUT_F), jnp.float32)],
    )(xt, wt, b2d, g2d, beta2d)

    h = pl.pallas_call(
        _forward_kernel,
        grid=(P_PAD // FBK,),
        in_specs=[pl.BlockSpec((IN_F, FBK), lambda i: (0, i)),
                  pl.BlockSpec((IN_F, OUT_F), lambda i: (0, 0)),
                  pl.BlockSpec((1, OUT_F), lambda i: (0, 0))],
        out_specs=pl.BlockSpec((FBK, OUT_F), lambda i: (i, 0)),
        out_shape=jax.ShapeDtypeStruct((P_PAD, OUT_F), jnp.float32),
    )(xt, w2, b2)

    mesh = plsc.VectorSubcoreMesh(core_axis_name="c", subcore_axis_name="s")
    grid_flat = pl.kernel(
        _scatter_kernel,
        mesh=mesh,
        out_type=jax.ShapeDtypeStruct((N_PX * N_PY, 2 * OUT_F),
                                      jnp.float32),
        compiler_params=pltpu.CompilerParams(use_tc_tiling_on_sc=False,
                                             needs_layout_passes=False),
        scratch_types=[
            pltpu.VMEM((LIST_CAP,), jnp.int32),
            pltpu.VMEM((PT,), jnp.int32),
            pltpu.VMEM((16 * NBUCKETS,), jnp.int32),
            pltpu.VMEM((16 * NBUCKETS,), jnp.int32),
            pltpu.VMEM((NBUCKETS,), jnp.int32),
            pltpu.VMEM((NBUCKETS,), jnp.int32),
            pltpu.VMEM((WAVE,), jnp.int32),
            pltpu.VMEM((WAVE,), jnp.int32),
            pltpu.VMEM((WAVE, OUT_F), jnp.float32),
            pltpu.VMEM((BIGWAVE,), jnp.int32),
            pltpu.VMEM((BIGWAVE,), jnp.int32),
            pltpu.VMEM((BIGWAVE, OUT_F), jnp.float32),
            pltpu.VMEM((BIGWAVE,), jnp.int32),
            pltpu.VMEM((BIGWAVE,), jnp.int32),
            pltpu.VMEM((BIGWAVE, OUT_F), jnp.float32),
            pltpu.SemaphoreType.DMA,
            pltpu.SemaphoreType.DMA,
            pltpu.SemaphoreType.DMA,
            pltpu.SemaphoreType.DMA,
            pltpu.VMEM((128, OUT_F), jnp.float32),
            pltpu.VMEM_SHARED((CHUNK + 8, OUT_F), jnp.float32),
        ],
    )(xi, yi, h)

    return grid_flat[:, :OUT_F].reshape(N_PX, N_PY, OUT_F)
